# Initial kernel scaffold; baseline (speedup 1.0000x reference)
import functools

import jax
import jax.numpy as jnp
from jax import lax
from jax.experimental import pallas as pl
from jax.experimental.pallas import tpu as pltpu
from jax.experimental.pallas import tpu_sc as plsc

D = 128
N = 10000
E = 160000
T = 160000
NC = 2
NS = 16
NW = NC * NS

_mesh = functools.partial(plsc.VectorSubcoreMesh,
                          core_axis_name="c", subcore_axis_name="s")


def _silu(x):
    return x * jax.nn.sigmoid(x)


def _mm(x, w):
    return jax.lax.dot_general(x, w, (((1,), (0,)), ((), ())),
                               preferred_element_type=jnp.float32)


def _tc_node_mlp(h, W, b):
    BLK = 1000

    def body(h_ref, w_ref, b_ref, o_ref):
        o_ref[...] = _silu(_mm(h_ref[...], w_ref[...]) + b_ref[...])

    return pl.pallas_call(
        body,
        grid=(N // BLK,),
        in_specs=[pl.BlockSpec((BLK, D), lambda i: (i, 0)),
                  pl.BlockSpec((D, D), lambda i: (0, 0)),
                  pl.BlockSpec((1, D), lambda i: (0, 0))],
        out_specs=pl.BlockSpec((BLK, D), lambda i: (i, 0)),
        out_shape=jax.ShapeDtypeStruct((N, D), jnp.float32),
    )(h, W, b)


def _tc_sbro(sbf1, sbf2, rbf, s1a, s1ab, s1b, s1bb, s2a, s2ab, s2b, s2bb, ro_W):
    BLK = 2000

    def body(x1, x2, rb, wa1, ba1, wb1, bb1, wa2, ba2, wb2, bb2, wro,
             o1, o2, oro):
        o1[...] = _silu(_mm(_silu(_mm(x1[...], wa1[...]) + ba1[...]),
                            wb1[...]) + bb1[...])
        o2[...] = _silu(_mm(_silu(_mm(x2[...], wa2[...]) + ba2[...]),
                            wb2[...]) + bb2[...])
        oro[...] = _mm(rb[...], wro[...])

    full = pl.BlockSpec((D, D), lambda i: (0, 0))
    bias = pl.BlockSpec((1, D), lambda i: (0, 0))
    blk = pl.BlockSpec((BLK, D), lambda i: (i, 0))
    return pl.pallas_call(
        body,
        grid=(T // BLK,),
        in_specs=[blk, blk, blk, full, bias, full, bias, full, bias, full,
                  bias, full],
        out_specs=[blk, blk, blk],
        out_shape=[jax.ShapeDtypeStruct((T, D), jnp.float32),
                   jax.ShapeDtypeStruct((T, D), jnp.float32),
                   jax.ShapeDtypeStruct((E, D), jnp.float32)],
    )(sbf1, sbf2, rbf, s1a, s1ab, s1b, s1bb, s2a, s2ab, s2b, s2bb, ro_W)


def _tc_edge1(hi, hj, rbf, kA, kB, kC, kb, r1W, jA, jB, jC, jb):
    BLK = 2000

    def body(hi_r, hj_r, rb_r, kA_r, kB_r, kC_r, kb_r, r1_r, jA_r, jB_r,
             jC_r, jb_r, okj, oji):
        hi_x, hj_x, rb_x = hi_r[...], hj_r[...], rb_r[...]
        pre_k = (_mm(hi_x, kA_r[...]) + _mm(hj_x, kB_r[...])
                 + _mm(rb_x, kC_r[...]) + kb_r[...])
        okj[...] = _silu(pre_k) * _mm(rb_x, r1_r[...])
        pre_j = (_mm(hi_x, jA_r[...]) + _mm(hj_x, jB_r[...])
                 + _mm(rb_x, jC_r[...]) + jb_r[...])
        oji[...] = _silu(pre_j)

    full = pl.BlockSpec((D, D), lambda i: (0, 0))
    bias = pl.BlockSpec((1, D), lambda i: (0, 0))
    blk = pl.BlockSpec((BLK, D), lambda i: (i, 0))
    return pl.pallas_call(
        body,
        grid=(E // BLK,),
        in_specs=[blk, blk, blk, full, full, full, bias, full, full, full,
                  full, bias],
        out_specs=[blk, blk],
        out_shape=[jax.ShapeDtypeStruct((E, D), jnp.float32),
                   jax.ShapeDtypeStruct((E, D), jnp.float32)],
    )(hi, hj, rbf, kA, kB, kC, kb, r1W, jA, jB, jC, jb)


def _tc_edge2(m1, rbf, jjW, jjb, r2W, j2W, j2b):
    BLK = 2000

    def body(m_r, rb_r, jjW_r, jjb_r, r2_r, j2W_r, j2b_r, ojj, oji):
        m_x, rb_x = m_r[...], rb_r[...]
        ojj[...] = _silu(_mm(m_x, jjW_r[...]) + jjb_r[...]) * _mm(rb_x, r2_r[...])
        oji[...] = _silu(_mm(m_x, j2W_r[...]) + j2b_r[...])

    full = pl.BlockSpec((D, D), lambda i: (0, 0))
    bias = pl.BlockSpec((1, D), lambda i: (0, 0))
    blk = pl.BlockSpec((BLK, D), lambda i: (i, 0))
    return pl.pallas_call(
        body,
        grid=(E // BLK,),
        in_specs=[blk, blk, full, bias, full, full, bias],
        out_specs=[blk, blk],
        out_shape=[jax.ShapeDtypeStruct((E, D), jnp.float32),
                   jax.ShapeDtypeStruct((E, D), jnp.float32)],
    )(m1, rbf, jjW, jjb, r2W, j2W, j2b)


def _tc_final(p0, p1, h, ws):
    BLK = 1000

    def body(p0_r, p1_r, h_r,
             r1a, r1ab, r1b, r1bb, hW, hb, r2a, r2ab, r2b, r2bb,
             r3a, r3ab, r3b, r3bb, y1, y1b, y2, y2b, y3, y3b, yW, ybp,
             oh, oy):
        x = p0_r[...] + p1_r[...]

        def res(x, wa, ba, wb, bb):
            return _silu(_mm(_silu(_mm(x, wa[...]) + ba[...]), wb[...])
                         + bb[...]) + x

        x = res(x, r1a, r1ab, r1b, r1bb)
        x = _silu(_mm(x, hW[...]) + hb[...]) + h_r[...]
        x = res(x, r2a, r2ab, r2b, r2bb)
        x = res(x, r3a, r3ab, r3b, r3bb)
        oh[...] = x
        t = _silu(_mm(x, y1[...]) + y1b[...])
        t = _silu(_mm(t, y2[...]) + y2b[...])
        t = _silu(_mm(t, y3[...]) + y3b[...])
        oy[...] = _mm(t, yW[...]) + ybp[...]

    full = pl.BlockSpec((D, D), lambda i: (0, 0))
    bias = pl.BlockSpec((1, D), lambda i: (0, 0))
    blk = pl.BlockSpec((BLK, D), lambda i: (i, 0))
    return pl.pallas_call(
        body,
        grid=(N // BLK,),
        in_specs=[blk, blk, blk] + [full, bias] * 11,
        out_specs=[blk, blk],
        out_shape=[jax.ShapeDtypeStruct((N, D), jnp.float32),
                   jax.ShapeDtypeStruct((N, D), jnp.float32)],
    )(p0, p1, h, *ws)


def _sc_gather2(h1, ii, jj):
    B = 128
    NBLK = E // B

    @functools.partial(
        pl.kernel, mesh=_mesh(),
        out_type=[jax.ShapeDtypeStruct((E, D), jnp.float32),
                  jax.ShapeDtypeStruct((E, D), jnp.float32)],
        scratch_types=[pltpu.VMEM((B,), jnp.int32),
                       pltpu.VMEM((B,), jnp.int32),
                       pltpu.VMEM((B, D), jnp.float32),
                       pltpu.VMEM((B, D), jnp.float32),
                       pltpu.SemaphoreType.DMA,
                       pltpu.SemaphoreType.DMA],
    )
    def k(h1_hbm, ii_hbm, jj_hbm, hi_hbm, hj_hbm, iv, jv, ri, rj, s1, s2):
        wid = lax.axis_index("s") * NC + lax.axis_index("c")
        nblk = NBLK // NW + jnp.where(wid < NBLK % NW, 1, 0)

        def blk(q, _):
            off = (wid + q * NW) * B
            pltpu.sync_copy(ii_hbm.at[pl.ds(off, B)], iv)
            pltpu.sync_copy(jj_hbm.at[pl.ds(off, B)], jv)
            c1 = pltpu.async_copy(h1_hbm.at[iv], ri, s1)
            c2 = pltpu.async_copy(h1_hbm.at[jv], rj, s2)
            c1.wait()
            c2.wait()
            c3 = pltpu.async_copy(ri, hi_hbm.at[pl.ds(off, B)], s1)
            c4 = pltpu.async_copy(rj, hj_hbm.at[pl.ds(off, B)], s2)
            c3.wait()
            c4.wait()
            return 0

        lax.fori_loop(0, nblk, blk, 0)

    return k(h1, ii, jj)


def _sc_agg(mji, mkj, sb, jid, kid):
    C = 8000
    KC = E // C
    KSC = KC // NC
    TS = T // NS
    NV = TS // 16
    B = 128
    MAXB = (TS + B - 1) // B + 1
    RW = C // NS

    @functools.partial(
        pl.kernel, mesh=_mesh(),
        out_type=jax.ShapeDtypeStruct((E, D), jnp.float32),
        scratch_types=[
            pltpu.VMEM((TS,), jnp.int32),
            pltpu.VMEM((TS,), jnp.int32),
            pltpu.VMEM((MAXB, B), jnp.int32),
            pltpu.VMEM((MAXB, B), jnp.int32),
            pltpu.VMEM((MAXB, B), jnp.int32),
            pltpu.VMEM((B, D), jnp.float32),
            pltpu.VMEM((B, D), jnp.float32),
            pltpu.VMEM_SHARED((C + 16, D), jnp.float32),
            pltpu.SemaphoreType.DMA,
            pltpu.SemaphoreType.DMA,
        ],
    )
    def k(mji_hbm, mkj_hbm, sb_hbm, jid_hbm, kid_hbm, out_hbm,
          jv_ref, kv_ref, cbk, cbt, cbd, rowsA, rowsB, acc, s1, s2):
        cid = lax.axis_index("c")
        sid = lax.axis_index("s")
        wid = sid * NC + cid
        tbase = sid * TS
        pltpu.sync_copy(jid_hbm.at[pl.ds(tbase, TS)], jv_ref)
        pltpu.sync_copy(kid_hbm.at[pl.ds(tbase, TS)], kv_ref)
        lanes = lax.iota(jnp.int32, 16)

        @pl.loop(0, KSC)
        def chunk(kc):
            cbase = (cid * KSC + kc) * C
            pltpu.sync_copy(mji_hbm.at[pl.ds(cbase + sid * RW, RW)],
                            acc.at[pl.ds(sid * RW, RW)])
            plsc.subcore_barrier()

            def scan_body(v, off):
                jx = jv_ref[pl.ds(v * 16, 16)]
                kx = kv_ref[pl.ds(v * 16, 16)]
                lj = jx - cbase
                msk = (lj >= 0) & (lj < C)
                ones = msk.astype(jnp.int32)
                inc = plsc.cumsum(ones)
                tot = jnp.sum(ones)
                pos = off + inc - 1
                prow = lax.shift_right_logical(pos, 7)
                pcol = lax.bitwise_and(pos, B - 1)
                tvec = tbase + v * 16 + lanes
                plsc.store_scatter(cbk, [prow, pcol], kx, mask=msk)
                plsc.store_scatter(cbt, [prow, pcol], tvec, mask=msk)
                plsc.store_scatter(cbd, [prow, pcol], lj, mask=msk)
                return off + tot

            off = lax.fori_loop(0, NV, scan_body, jnp.int32(0))
            nb = lax.shift_right_logical(off + B - 1, 7)
            pstart = lax.shift_right_logical(off, 4) * 16
            npv = lax.shift_right_logical(nb * B - pstart, 4)

            def pad_body(q, _):
                pos = pstart + q * 16 + lanes
                mskp = pos >= off
                prow = lax.shift_right_logical(pos, 7)
                pcol = lax.bitwise_and(pos, B - 1)
                padk = wid * 16 + lanes
                padd = C + lanes
                plsc.store_scatter(cbk, [prow, pcol], padk, mask=mskp)
                plsc.store_scatter(cbt, [prow, pcol], padk, mask=mskp)
                plsc.store_scatter(cbd, [prow, pcol], padd, mask=mskp)
                return 0

            lax.fori_loop(0, npv, pad_body, 0)

            def blk_body(b, _):
                c1 = pltpu.async_copy(mkj_hbm.at[cbk.at[b]], rowsA, s1)
                c2 = pltpu.async_copy(sb_hbm.at[cbt.at[b]], rowsB, s2)
                c1.wait()
                c2.wait()

                @pl.loop(0, B)
                def mul_row(r):
                    for cc in range(0, D, 16):
                        rowsA[r, pl.ds(cc, 16)] = (rowsA[r, pl.ds(cc, 16)]
                                                   * rowsB[r, pl.ds(cc, 16)])

                pltpu.sync_copy(rowsA, acc.at[cbd.at[b]], add=True)
                return 0

            lax.fori_loop(0, nb, blk_body, 0)
            plsc.subcore_barrier()
            pltpu.sync_copy(acc.at[pl.ds(sid * RW, RW)],
                            out_hbm.at[pl.ds(cbase + sid * RW, RW)])
            plsc.subcore_barrier()

    return k(mji, mkj, sb, jid, kid)


def _sc_nagg(m2, ro, iidx):
    B = 128
    NBLK = E // B
    ZB = 125
    ZR = N // NS

    @functools.partial(
        pl.kernel, mesh=_mesh(),
        out_type=[jax.ShapeDtypeStruct((N, D), jnp.float32),
                  jax.ShapeDtypeStruct((N, D), jnp.float32)],
        scratch_types=[
            pltpu.VMEM((ZB, D), jnp.float32),
            pltpu.VMEM((B, D), jnp.float32),
            pltpu.VMEM((B, D), jnp.float32),
            pltpu.VMEM((B,), jnp.int32),
            pltpu.VMEM_SHARED((N, D), jnp.float32),
            pltpu.SemaphoreType.DMA,
            pltpu.SemaphoreType.DMA,
        ],
    )
    def k(m2_hbm, ro_hbm, i_hbm, p0_hbm, p1_hbm,
          zbuf, bufA, bufB, ibuf, acc, s1, s2):
        cid = lax.axis_index("c")
        sid = lax.axis_index("s")
        wid = sid * NC + cid

        @pl.loop(0, ZB)
        def zrow(r):
            for cc in range(0, D, 16):
                zbuf[r, pl.ds(cc, 16)] = jnp.zeros((16,), jnp.float32)

        @pl.loop(0, ZR // ZB)
        def zcopy(z):
            pltpu.sync_copy(zbuf, acc.at[pl.ds(sid * ZR + z * ZB, ZB)])

        plsc.subcore_barrier()
        nblk = NBLK // NW + jnp.where(wid < NBLK % NW, 1, 0)

        def blk(q, _):
            off = (wid + q * NW) * B
            c1 = pltpu.async_copy(m2_hbm.at[pl.ds(off, B)], bufA, s1)
            c2 = pltpu.async_copy(ro_hbm.at[pl.ds(off, B)], bufB, s2)
            pltpu.sync_copy(i_hbm.at[pl.ds(off, B)], ibuf)
            c1.wait()
            c2.wait()

            @pl.loop(0, B)
            def mul_row(r):
                for cc in range(0, D, 16):
                    bufA[r, pl.ds(cc, 16)] = (bufA[r, pl.ds(cc, 16)]
                                              * bufB[r, pl.ds(cc, 16)])

            pltpu.sync_copy(bufA, acc.at[ibuf], add=True)
            return 0

        lax.fori_loop(0, nblk, blk, 0)
        plsc.subcore_barrier()

        @pl.when(cid == 0)
        def w0():
            pltpu.sync_copy(acc.at[pl.ds(sid * ZR, ZR)],
                            p0_hbm.at[pl.ds(sid * ZR, ZR)])

        @pl.when(cid == 1)
        def w1():
            pltpu.sync_copy(acc.at[pl.ds(sid * ZR, ZR)],
                            p1_hbm.at[pl.ds(sid * ZR, ZR)])

    return k(m2, ro, iidx)


def kernel(h, rbf, sbf1, sbf2, idx_kj, idx_ji_1, idx_jj, idx_ji_2,
           edge_index, params):
    p = params
    i32 = jnp.int32
    ii = edge_index[1].astype(i32)
    jj = edge_index[0].astype(i32)
    idx_kj = idx_kj.astype(i32)
    idx_ji_1 = idx_ji_1.astype(i32)
    idx_jj = idx_jj.astype(i32)
    idx_ji_2 = idx_ji_2.astype(i32)

    def b2(name):
        return p[name].reshape(1, D)

    kW, jW = p['kj_W'], p['ji1_W']

    h1 = _tc_node_mlp(h, p['h_W'], b2('h_b'))
    hi, hj = _sc_gather2(h1, ii, jj)
    sb1, sb2, ro = _tc_sbro(sbf1, sbf2, rbf,
                            p['s1a_W'], b2('s1a_b'), p['s1b_W'], b2('s1b_b'),
                            p['s2a_W'], b2('s2a_b'), p['s2b_W'], b2('s2b_b'),
                            p['rbfo_W'])
    m_kj, m_ji1 = _tc_edge1(hi, hj, rbf,
                            kW[:D], kW[D:2 * D], kW[2 * D:], b2('kj_b'),
                            p['rbf1_W'],
                            jW[:D], jW[D:2 * D], jW[2 * D:], b2('ji1_b'))
    m1 = _sc_agg(m_ji1, m_kj, sb1, idx_ji_1, idx_kj)
    m_jj, m_ji2 = _tc_edge2(m1, rbf, p['jj_W'], b2('jj_b'), p['rbf2_W'],
                            p['ji2_W'], b2('ji2_b'))
    m2 = _sc_agg(m_ji2, m_jj, sb2, idx_ji_2, idx_jj)
    p0, p1 = _sc_nagg(m2, ro, ii)

    yWpad = jnp.pad(p['yW_W'], ((0, 0), (0, D - 1)))
    ybpad = jnp.pad(p['yW_b'], (0, D - 1)).reshape(1, D)
    ws = [p['r1a_W'], b2('r1a_b'), p['r1b_W'], b2('r1b_b'),
          p['h_W'], b2('h_b'),
          p['r2a_W'], b2('r2a_b'), p['r2b_W'], b2('r2b_b'),
          p['r3a_W'], b2('r3a_b'), p['r3b_W'], b2('r3b_b'),
          p['y1_W'], b2('y1_b'), p['y2_W'], b2('y2_b'),
          p['y3_W'], b2('y3_b'), yWpad, ybpad]
    h2, yfull = _tc_final(p0, p1, h, ws)
    return (h2, yfull[:, :1])


# speedup vs baseline: 1.7018x; 1.7018x over previous
import dataclasses
import functools

import jax
import jax.numpy as jnp
from jax import lax
from jax.experimental import pallas as pl
from jax.experimental.pallas import tpu as pltpu
from jax.experimental.pallas import tpu_sc as plsc

D = 128
N = 10000
E = 160000
T = 160000
NC = 2
NS = 16
NW = NC * NS

_mesh = functools.partial(plsc.VectorSubcoreMesh,
                          core_axis_name="c", subcore_axis_name="s")

_sc_params = pltpu.CompilerParams()
if "needs_layout_passes" in pltpu.CompilerParams.__dataclass_fields__:
    _sc_params = dataclasses.replace(_sc_params, needs_layout_passes=False)


def _silu(x):
    return x * jax.nn.sigmoid(x)


def _mm(x, w):
    return jax.lax.dot_general(x, w, (((1,), (0,)), ((), ())),
                               preferred_element_type=jnp.float32)


def _tc_node_mlp(h, W, b):
    BLK = 1000

    def body(h_ref, w_ref, b_ref, o_ref):
        o_ref[...] = _silu(_mm(h_ref[...], w_ref[...]) + b_ref[...])

    return pl.pallas_call(
        body,
        grid=(N // BLK,),
        in_specs=[pl.BlockSpec((BLK, D), lambda i: (i, 0)),
                  pl.BlockSpec((D, D), lambda i: (0, 0)),
                  pl.BlockSpec((1, D), lambda i: (0, 0))],
        out_specs=pl.BlockSpec((BLK, D), lambda i: (i, 0)),
        out_shape=jax.ShapeDtypeStruct((N, D), jnp.float32),
    )(h, W, b)


def _tc_sbro(sbf1, sbf2, rbf, s1a, s1ab, s1b, s1bb, s2a, s2ab, s2b, s2bb, ro_W):
    BLK = 2000

    def body(x1, x2, rb, wa1, ba1, wb1, bb1, wa2, ba2, wb2, bb2, wro,
             o1, o2, oro):
        o1[...] = _silu(_mm(_silu(_mm(x1[...], wa1[...]) + ba1[...]),
                            wb1[...]) + bb1[...])
        o2[...] = _silu(_mm(_silu(_mm(x2[...], wa2[...]) + ba2[...]),
                            wb2[...]) + bb2[...])
        oro[...] = _mm(rb[...], wro[...])

    full = pl.BlockSpec((D, D), lambda i: (0, 0))
    bias = pl.BlockSpec((1, D), lambda i: (0, 0))
    blk = pl.BlockSpec((BLK, D), lambda i: (i, 0))
    return pl.pallas_call(
        body,
        grid=(T // BLK,),
        in_specs=[blk, blk, blk, full, bias, full, bias, full, bias, full,
                  bias, full],
        out_specs=[blk, blk, blk],
        out_shape=[jax.ShapeDtypeStruct((T, D), jnp.float32),
                   jax.ShapeDtypeStruct((T, D), jnp.float32),
                   jax.ShapeDtypeStruct((E, D), jnp.float32)],
    )(sbf1, sbf2, rbf, s1a, s1ab, s1b, s1bb, s2a, s2ab, s2b, s2bb, ro_W)


def _tc_edge1(hi, hj, rbf, kA, kB, kC, kb, r1W, jA, jB, jC, jb):
    BLK = 2000

    def body(hi_r, hj_r, rb_r, kA_r, kB_r, kC_r, kb_r, r1_r, jA_r, jB_r,
             jC_r, jb_r, okj, oji):
        hi_x, hj_x, rb_x = hi_r[...], hj_r[...], rb_r[...]
        pre_k = (_mm(hi_x, kA_r[...]) + _mm(hj_x, kB_r[...])
                 + _mm(rb_x, kC_r[...]) + kb_r[...])
        okj[...] = _silu(pre_k) * _mm(rb_x, r1_r[...])
        pre_j = (_mm(hi_x, jA_r[...]) + _mm(hj_x, jB_r[...])
                 + _mm(rb_x, jC_r[...]) + jb_r[...])
        oji[...] = _silu(pre_j)

    full = pl.BlockSpec((D, D), lambda i: (0, 0))
    bias = pl.BlockSpec((1, D), lambda i: (0, 0))
    blk = pl.BlockSpec((BLK, D), lambda i: (i, 0))
    return pl.pallas_call(
        body,
        grid=(E // BLK,),
        in_specs=[blk, blk, blk, full, full, full, bias, full, full, full,
                  full, bias],
        out_specs=[blk, blk],
        out_shape=[jax.ShapeDtypeStruct((E, D), jnp.float32),
                   jax.ShapeDtypeStruct((E, D), jnp.float32)],
    )(hi, hj, rbf, kA, kB, kC, kb, r1W, jA, jB, jC, jb)


def _tc_edge2(m1, rbf, jjW, jjb, r2W, j2W, j2b):
    BLK = 2000

    def body(m_r, rb_r, jjW_r, jjb_r, r2_r, j2W_r, j2b_r, ojj, oji):
        m_x, rb_x = m_r[...], rb_r[...]
        ojj[...] = _silu(_mm(m_x, jjW_r[...]) + jjb_r[...]) * _mm(rb_x, r2_r[...])
        oji[...] = _silu(_mm(m_x, j2W_r[...]) + j2b_r[...])

    full = pl.BlockSpec((D, D), lambda i: (0, 0))
    bias = pl.BlockSpec((1, D), lambda i: (0, 0))
    blk = pl.BlockSpec((BLK, D), lambda i: (i, 0))
    return pl.pallas_call(
        body,
        grid=(E // BLK,),
        in_specs=[blk, blk, full, bias, full, full, bias],
        out_specs=[blk, blk],
        out_shape=[jax.ShapeDtypeStruct((E, D), jnp.float32),
                   jax.ShapeDtypeStruct((E, D), jnp.float32)],
    )(m1, rbf, jjW, jjb, r2W, j2W, j2b)


def _tc_final(praw, h, ws):
    BLK = 1000

    def body(p_r, h_r,
             r1a, r1ab, r1b, r1bb, hW, hb, r2a, r2ab, r2b, r2bb,
             r3a, r3ab, r3b, r3bb, y1, y1b, y2, y2b, y3, y3b, yW, ybp,
             oh, oy):
        x = p_r[...]

        def res(x, wa, ba, wb, bb):
            return _silu(_mm(_silu(_mm(x, wa[...]) + ba[...]), wb[...])
                         + bb[...]) + x

        x = res(x, r1a, r1ab, r1b, r1bb)
        x = _silu(_mm(x, hW[...]) + hb[...]) + h_r[...]
        x = res(x, r2a, r2ab, r2b, r2bb)
        x = res(x, r3a, r3ab, r3b, r3bb)
        oh[...] = x
        t = _silu(_mm(x, y1[...]) + y1b[...])
        t = _silu(_mm(t, y2[...]) + y2b[...])
        t = _silu(_mm(t, y3[...]) + y3b[...])
        oy[...] = _mm(t, yW[...]) + ybp[...]

    full = pl.BlockSpec((D, D), lambda i: (0, 0))
    bias = pl.BlockSpec((1, D), lambda i: (0, 0))
    blk = pl.BlockSpec((BLK, D), lambda i: (i, 0))
    return pl.pallas_call(
        body,
        grid=(N // BLK,),
        in_specs=[blk, blk] + [full, bias] * 11,
        out_specs=[blk, blk],
        out_shape=[jax.ShapeDtypeStruct((N, D), jnp.float32),
                   jax.ShapeDtypeStruct((N, D), jnp.float32)],
    )(praw, h, *ws)


def _sc_gather2(h1, ii, jj):
    B = 128
    NBLK = E // B

    @functools.partial(
        pl.kernel, mesh=_mesh(), compiler_params=_sc_params,
        out_type=[jax.ShapeDtypeStruct((E, D), jnp.float32),
                  jax.ShapeDtypeStruct((E, D), jnp.float32)],
        scratch_types=[pltpu.VMEM((B,), jnp.int32),
                       pltpu.VMEM((B,), jnp.int32),
                       pltpu.VMEM((B, D), jnp.float32),
                       pltpu.VMEM((B, D), jnp.float32),
                       pltpu.SemaphoreType.DMA,
                       pltpu.SemaphoreType.DMA],
    )
    def k(h1_hbm, ii_hbm, jj_hbm, hi_hbm, hj_hbm, iv, jv, ri, rj, s1, s2):
        wid = lax.axis_index("s") * NC + lax.axis_index("c")
        nblk = NBLK // NW + jnp.where(wid < NBLK % NW, 1, 0)

        def blk(q, _):
            off = (wid + q * NW) * B
            pltpu.sync_copy(ii_hbm.at[pl.ds(off, B)], iv)
            pltpu.sync_copy(jj_hbm.at[pl.ds(off, B)], jv)
            c1 = pltpu.async_copy(h1_hbm.at[iv], ri, s1)
            c2 = pltpu.async_copy(h1_hbm.at[jv], rj, s2)
            c1.wait()
            c2.wait()
            c3 = pltpu.async_copy(ri, hi_hbm.at[pl.ds(off, B)], s1)
            c4 = pltpu.async_copy(rj, hj_hbm.at[pl.ds(off, B)], s2)
            c3.wait()
            c4.wait()
            return 0

        lax.fori_loop(0, nblk, blk, 0)

    return k(h1, ii, jj)


def _sc_agg(mji, mkj, sb, jid, kid):
    C = 3200
    KC = E // C
    KSC = KC // NC
    TS = T // NS
    NV = TS // 16
    B = 128
    MAXB = (TS + B - 1) // B + 1
    RW = C // NS

    @functools.partial(
        pl.kernel, mesh=_mesh(), compiler_params=_sc_params,
        out_type=jax.ShapeDtypeStruct((E, D), jnp.float32),
        scratch_types=[
            pltpu.VMEM((TS,), jnp.int32),
            pltpu.VMEM((TS,), jnp.int32),
            pltpu.VMEM((MAXB, B), jnp.int32),
            pltpu.VMEM((MAXB, B), jnp.int32),
            pltpu.VMEM((MAXB, B), jnp.int32),
            pltpu.VMEM((B, D), jnp.float32),
            pltpu.VMEM((B, D), jnp.float32),
            pltpu.VMEM_SHARED((C + 16, D), jnp.float32),
            pltpu.SemaphoreType.DMA,
            pltpu.SemaphoreType.DMA,
        ],
    )
    def k(mji_hbm, mkj_hbm, sb_hbm, jid_hbm, kid_hbm, out_hbm,
          jv_ref, kv_ref, cbk, cbt, cbd, rowsA, rowsB, acc, s1, s2):
        cid = lax.axis_index("c")
        sid = lax.axis_index("s")
        wid = sid * NC + cid
        tbase = sid * TS
        pltpu.sync_copy(jid_hbm.at[pl.ds(tbase, TS)], jv_ref)
        pltpu.sync_copy(kid_hbm.at[pl.ds(tbase, TS)], kv_ref)
        lanes = lax.iota(jnp.int32, 16)
        ck_lo = cid * KSC

        def chunk(ck, _):
            cbase = ck * C
            pltpu.sync_copy(mji_hbm.at[pl.ds(cbase + sid * RW, RW)],
                            acc.at[pl.ds(sid * RW, RW)])
            plsc.subcore_barrier()

            def scan_body(v, off):
                jx = jv_ref[pl.ds(v * 16, 16)]
                kx = kv_ref[pl.ds(v * 16, 16)]
                lj = jx - cbase
                msk = (lj >= 0) & (lj < C)
                ones = msk.astype(jnp.int32)
                inc = plsc.cumsum(ones)
                tot = jnp.sum(ones)
                pos = off + inc - 1
                prow = lax.shift_right_logical(pos, 7)
                pcol = lax.bitwise_and(pos, B - 1)
                tvec = tbase + v * 16 + lanes
                plsc.store_scatter(cbk, [prow, pcol], kx, mask=msk)
                plsc.store_scatter(cbt, [prow, pcol], tvec, mask=msk)
                plsc.store_scatter(cbd, [prow, pcol], lj, mask=msk)
                return off + tot

            off = lax.fori_loop(0, NV, scan_body, jnp.int32(0))
            nb = lax.shift_right_logical(off + B - 1, 7)
            pstart = lax.shift_right_logical(off, 4) * 16
            npv = lax.shift_right_logical(nb * B - pstart, 4)

            def pad_body(q, _):
                pos = pstart + q * 16 + lanes
                mskp = pos >= off
                prow = lax.shift_right_logical(pos, 7)
                pcol = lax.bitwise_and(pos, B - 1)
                padk = wid * 16 + lanes
                padd = C + lanes
                plsc.store_scatter(cbk, [prow, pcol], padk, mask=mskp)
                plsc.store_scatter(cbt, [prow, pcol], padk, mask=mskp)
                plsc.store_scatter(cbd, [prow, pcol], padd, mask=mskp)
                return 0

            lax.fori_loop(0, npv, pad_body, 0)

            def blk_body(b, _):
                c1 = pltpu.async_copy(mkj_hbm.at[cbk.at[b]], rowsA, s1)
                c2 = pltpu.async_copy(sb_hbm.at[cbt.at[b]], rowsB, s2)
                c1.wait()
                c2.wait()

                @pl.loop(0, B)
                def mul_row(r):
                    for cc in range(0, D, 16):
                        rowsA[r, pl.ds(cc, 16)] = (rowsA[r, pl.ds(cc, 16)]
                                                   * rowsB[r, pl.ds(cc, 16)])

                pltpu.sync_copy(rowsA, acc.at[cbd.at[b]], add=True)
                return 0

            lax.fori_loop(0, nb, blk_body, 0)
            plsc.subcore_barrier()
            pltpu.sync_copy(acc.at[pl.ds(sid * RW, RW)],
                            out_hbm.at[pl.ds(cbase + sid * RW, RW)])
            plsc.subcore_barrier()
            return 0

        lax.fori_loop(ck_lo, ck_lo + KSC, chunk, 0)

    return k(mji, mkj, sb, jid, kid)


def _sc_nagg(m2, ro, iidx):
    C0 = 5120
    ES = E // NS
    NV = ES // 16
    B = 128
    MAXB = (ES + B - 1) // B + 1
    ZB = 16

    @functools.partial(
        pl.kernel, mesh=_mesh(), compiler_params=_sc_params,
        out_type=jax.ShapeDtypeStruct((N, D), jnp.float32),
        scratch_types=[
            pltpu.VMEM((ES,), jnp.int32),
            pltpu.VMEM((MAXB, B), jnp.int32),
            pltpu.VMEM((MAXB, B), jnp.int32),
            pltpu.VMEM((B, D), jnp.float32),
            pltpu.VMEM((B, D), jnp.float32),
            pltpu.VMEM((ZB, D), jnp.float32),
            pltpu.VMEM_SHARED((C0 + 16, D), jnp.float32),
            pltpu.SemaphoreType.DMA,
            pltpu.SemaphoreType.DMA,
        ],
    )
    def k(m2_hbm, ro_hbm, i_hbm, out_hbm,
          iv_ref, cbe, cbd, rowsA, rowsB, zbuf, acc, s1, s2):
        cid = lax.axis_index("c")
        sid = lax.axis_index("s")
        wid = sid * NC + cid
        ebase = sid * ES
        pltpu.sync_copy(i_hbm.at[pl.ds(ebase, ES)], iv_ref)
        lanes = lax.iota(jnp.int32, 16)
        nbase = cid * C0
        climit = jnp.where(cid == 0, C0, N - C0)
        zoff = jnp.where(cid == 0, sid * 320, sid * 304)
        zn = jnp.where(cid == 0, 320,
                       jnp.where(sid < 15, 304, 320))

        @pl.loop(0, ZB)
        def zrow(r):
            for cc in range(0, D, 16):
                zbuf[r, pl.ds(cc, 16)] = jnp.zeros((16,), jnp.float32)

        def zcopy(z, _):
            pltpu.sync_copy(zbuf, acc.at[pl.ds(zoff + z * ZB, ZB)])
            return 0

        lax.fori_loop(0, zn // ZB, zcopy, 0)
        plsc.subcore_barrier()

        def scan_body(v, off):
            ix = iv_ref[pl.ds(v * 16, 16)]
            lj = ix - nbase
            msk = (lj >= 0) & (lj < climit)
            ones = msk.astype(jnp.int32)
            inc = plsc.cumsum(ones)
            tot = jnp.sum(ones)
            pos = off + inc - 1
            prow = lax.shift_right_logical(pos, 7)
            pcol = lax.bitwise_and(pos, B - 1)
            evec = ebase + v * 16 + lanes
            plsc.store_scatter(cbe, [prow, pcol], evec, mask=msk)
            plsc.store_scatter(cbd, [prow, pcol], lj, mask=msk)
            return off + tot

        off = lax.fori_loop(0, NV, scan_body, jnp.int32(0))
        nb = lax.shift_right_logical(off + B - 1, 7)
        pstart = lax.shift_right_logical(off, 4) * 16
        npv = lax.shift_right_logical(nb * B - pstart, 4)

        def pad_body(q, _):
            pos = pstart + q * 16 + lanes
            mskp = pos >= off
            prow = lax.shift_right_logical(pos, 7)
            pcol = lax.bitwise_and(pos, B - 1)
            pade = wid * 16 + lanes
            padd = C0 + lanes
            plsc.store_scatter(cbe, [prow, pcol], pade, mask=mskp)
            plsc.store_scatter(cbd, [prow, pcol], padd, mask=mskp)
            return 0

        lax.fori_loop(0, npv, pad_body, 0)

        def blk_body(b, _):
            c1 = pltpu.async_copy(m2_hbm.at[cbe.at[b]], rowsA, s1)
            c2 = pltpu.async_copy(ro_hbm.at[cbe.at[b]], rowsB, s2)
            c1.wait()
            c2.wait()

            @pl.loop(0, B)
            def mul_row(r):
                for cc in range(0, D, 16):
                    rowsA[r, pl.ds(cc, 16)] = (rowsA[r, pl.ds(cc, 16)]
                                               * rowsB[r, pl.ds(cc, 16)])

            pltpu.sync_copy(rowsA, acc.at[cbd.at[b]], add=True)
            return 0

        lax.fori_loop(0, nb, blk_body, 0)
        plsc.subcore_barrier()

        @pl.when(zn == 320)
        def wout():
            pltpu.sync_copy(acc.at[pl.ds(zoff, 320)],
                            out_hbm.at[pl.ds(nbase + zoff, 320)])

        @pl.when(zn == 304)
        def woutt():
            pltpu.sync_copy(acc.at[pl.ds(zoff, 304)],
                            out_hbm.at[pl.ds(nbase + zoff, 304)])

    return k(m2, ro, iidx)


def kernel(h, rbf, sbf1, sbf2, idx_kj, idx_ji_1, idx_jj, idx_ji_2,
           edge_index, params):
    p = params
    i32 = jnp.int32
    ii = edge_index[1].astype(i32)
    jj = edge_index[0].astype(i32)
    idx_kj = idx_kj.astype(i32)
    idx_ji_1 = idx_ji_1.astype(i32)
    idx_jj = idx_jj.astype(i32)
    idx_ji_2 = idx_ji_2.astype(i32)

    def b2(name):
        return p[name].reshape(1, D)

    kW, jW = p['kj_W'], p['ji1_W']

    h1 = _tc_node_mlp(h, p['h_W'], b2('h_b'))
    hi, hj = _sc_gather2(h1, ii, jj)
    sb1, sb2, ro = _tc_sbro(sbf1, sbf2, rbf,
                            p['s1a_W'], b2('s1a_b'), p['s1b_W'], b2('s1b_b'),
                            p['s2a_W'], b2('s2a_b'), p['s2b_W'], b2('s2b_b'),
                            p['rbfo_W'])
    m_kj, m_ji1 = _tc_edge1(hi, hj, rbf,
                            kW[:D], kW[D:2 * D], kW[2 * D:], b2('kj_b'),
                            p['rbf1_W'],
                            jW[:D], jW[D:2 * D], jW[2 * D:], b2('ji1_b'))
    m1 = _sc_agg(m_ji1, m_kj, sb1, idx_ji_1, idx_kj)
    m_jj, m_ji2 = _tc_edge2(m1, rbf, p['jj_W'], b2('jj_b'), p['rbf2_W'],
                            p['ji2_W'], b2('ji2_b'))
    m2 = _sc_agg(m_ji2, m_jj, sb2, idx_ji_2, idx_jj)
    h2raw = _sc_nagg(m2, ro, ii)

    yWpad = jnp.pad(p['yW_W'], ((0, 0), (0, D - 1)))
    ybpad = jnp.pad(p['yW_b'], (0, D - 1)).reshape(1, D)
    ws = [p['r1a_W'], b2('r1a_b'), p['r1b_W'], b2('r1b_b'),
          p['h_W'], b2('h_b'),
          p['r2a_W'], b2('r2a_b'), p['r2b_W'], b2('r2b_b'),
          p['r3a_W'], b2('r3a_b'), p['r3b_W'], b2('r3b_b'),
          p['y1_W'], b2('y1_b'), p['y2_W'], b2('y2_b'),
          p['y3_W'], b2('y3_b'), yWpad, ybpad]
    h2, yfull = _tc_final(h2raw, h, ws)
    return (h2, yfull[:, :1])


# speedup vs baseline: 1.7211x; 1.0113x over previous
import dataclasses
import functools

import jax
import jax.numpy as jnp
from jax import lax
from jax.experimental import pallas as pl
from jax.experimental.pallas import tpu as pltpu
from jax.experimental.pallas import tpu_sc as plsc

D = 128
N = 10000
E = 160000
T = 160000
NC = 2
NS = 16
NW = NC * NS

_mesh = functools.partial(plsc.VectorSubcoreMesh,
                          core_axis_name="c", subcore_axis_name="s")

_sc_params = pltpu.CompilerParams()
if "needs_layout_passes" in pltpu.CompilerParams.__dataclass_fields__:
    _sc_params = dataclasses.replace(_sc_params, needs_layout_passes=False)


def _silu(x):
    return x * jax.nn.sigmoid(x)


def _mm(x, w):
    return jax.lax.dot_general(x, w, (((1,), (0,)), ((), ())),
                               preferred_element_type=jnp.float32)


def _tc_node_mlp(h, W, b):
    BLK = 1000

    def body(h_ref, w_ref, b_ref, o_ref):
        o_ref[...] = _silu(_mm(h_ref[...], w_ref[...]) + b_ref[...])

    return pl.pallas_call(
        body,
        grid=(N // BLK,),
        in_specs=[pl.BlockSpec((BLK, D), lambda i: (i, 0)),
                  pl.BlockSpec((D, D), lambda i: (0, 0)),
                  pl.BlockSpec((1, D), lambda i: (0, 0))],
        out_specs=pl.BlockSpec((BLK, D), lambda i: (i, 0)),
        out_shape=jax.ShapeDtypeStruct((N, D), jnp.float32),
    )(h, W, b)


def _tc_sbro(sbf1, sbf2, rbf, s1a, s1ab, s1b, s1bb, s2a, s2ab, s2b, s2bb, ro_W):
    BLK = 2000

    def body(x1, x2, rb, wa1, ba1, wb1, bb1, wa2, ba2, wb2, bb2, wro,
             o1, o2, oro):
        o1[...] = _silu(_mm(_silu(_mm(x1[...], wa1[...]) + ba1[...]),
                            wb1[...]) + bb1[...])
        o2[...] = _silu(_mm(_silu(_mm(x2[...], wa2[...]) + ba2[...]),
                            wb2[...]) + bb2[...])
        oro[...] = _mm(rb[...], wro[...])

    full = pl.BlockSpec((D, D), lambda i: (0, 0))
    bias = pl.BlockSpec((1, D), lambda i: (0, 0))
    blk = pl.BlockSpec((BLK, D), lambda i: (i, 0))
    return pl.pallas_call(
        body,
        grid=(T // BLK,),
        in_specs=[blk, blk, blk, full, bias, full, bias, full, bias, full,
                  bias, full],
        out_specs=[blk, blk, blk],
        out_shape=[jax.ShapeDtypeStruct((T, D), jnp.float32),
                   jax.ShapeDtypeStruct((T, D), jnp.float32),
                   jax.ShapeDtypeStruct((E, D), jnp.float32)],
    )(sbf1, sbf2, rbf, s1a, s1ab, s1b, s1bb, s2a, s2ab, s2b, s2bb, ro_W)


def _tc_edge1(hi, hj, rbf, kA, kB, kC, kb, r1W, jA, jB, jC, jb):
    BLK = 2000

    def body(hi_r, hj_r, rb_r, kA_r, kB_r, kC_r, kb_r, r1_r, jA_r, jB_r,
             jC_r, jb_r, okj, oji):
        hi_x, hj_x, rb_x = hi_r[...], hj_r[...], rb_r[...]
        pre_k = (_mm(hi_x, kA_r[...]) + _mm(hj_x, kB_r[...])
                 + _mm(rb_x, kC_r[...]) + kb_r[...])
        okj[...] = _silu(pre_k) * _mm(rb_x, r1_r[...])
        pre_j = (_mm(hi_x, jA_r[...]) + _mm(hj_x, jB_r[...])
                 + _mm(rb_x, jC_r[...]) + jb_r[...])
        oji[...] = _silu(pre_j)

    full = pl.BlockSpec((D, D), lambda i: (0, 0))
    bias = pl.BlockSpec((1, D), lambda i: (0, 0))
    blk = pl.BlockSpec((BLK, D), lambda i: (i, 0))
    return pl.pallas_call(
        body,
        grid=(E // BLK,),
        in_specs=[blk, blk, blk, full, full, full, bias, full, full, full,
                  full, bias],
        out_specs=[blk, blk],
        out_shape=[jax.ShapeDtypeStruct((E, D), jnp.float32),
                   jax.ShapeDtypeStruct((E, D), jnp.float32)],
    )(hi, hj, rbf, kA, kB, kC, kb, r1W, jA, jB, jC, jb)


def _tc_edge2(m1, rbf, jjW, jjb, r2W, j2W, j2b):
    BLK = 2000

    def body(m_r, rb_r, jjW_r, jjb_r, r2_r, j2W_r, j2b_r, ojj, oji):
        m_x, rb_x = m_r[...], rb_r[...]
        ojj[...] = _silu(_mm(m_x, jjW_r[...]) + jjb_r[...]) * _mm(rb_x, r2_r[...])
        oji[...] = _silu(_mm(m_x, j2W_r[...]) + j2b_r[...])

    full = pl.BlockSpec((D, D), lambda i: (0, 0))
    bias = pl.BlockSpec((1, D), lambda i: (0, 0))
    blk = pl.BlockSpec((BLK, D), lambda i: (i, 0))
    return pl.pallas_call(
        body,
        grid=(E // BLK,),
        in_specs=[blk, blk, full, bias, full, full, bias],
        out_specs=[blk, blk],
        out_shape=[jax.ShapeDtypeStruct((E, D), jnp.float32),
                   jax.ShapeDtypeStruct((E, D), jnp.float32)],
    )(m1, rbf, jjW, jjb, r2W, j2W, j2b)


def _tc_final(praw, h, ws):
    BLK = 1000

    def body(p_r, h_r,
             r1a, r1ab, r1b, r1bb, hW, hb, r2a, r2ab, r2b, r2bb,
             r3a, r3ab, r3b, r3bb, y1, y1b, y2, y2b, y3, y3b, yW, ybp,
             oh, oy):
        x = p_r[...]

        def res(x, wa, ba, wb, bb):
            return _silu(_mm(_silu(_mm(x, wa[...]) + ba[...]), wb[...])
                         + bb[...]) + x

        x = res(x, r1a, r1ab, r1b, r1bb)
        x = _silu(_mm(x, hW[...]) + hb[...]) + h_r[...]
        x = res(x, r2a, r2ab, r2b, r2bb)
        x = res(x, r3a, r3ab, r3b, r3bb)
        oh[...] = x
        t = _silu(_mm(x, y1[...]) + y1b[...])
        t = _silu(_mm(t, y2[...]) + y2b[...])
        t = _silu(_mm(t, y3[...]) + y3b[...])
        oy[...] = _mm(t, yW[...]) + ybp[...]

    full = pl.BlockSpec((D, D), lambda i: (0, 0))
    bias = pl.BlockSpec((1, D), lambda i: (0, 0))
    blk = pl.BlockSpec((BLK, D), lambda i: (i, 0))
    return pl.pallas_call(
        body,
        grid=(N // BLK,),
        in_specs=[blk, blk] + [full, bias] * 11,
        out_specs=[blk, blk],
        out_shape=[jax.ShapeDtypeStruct((N, D), jnp.float32),
                   jax.ShapeDtypeStruct((N, D), jnp.float32)],
    )(praw, h, *ws)


def _sc_gather2(h1, ii, jj):
    B = 128
    EW = E // NW
    NF = EW // B
    TL = EW - NF * B

    @functools.partial(
        pl.kernel, mesh=_mesh(), compiler_params=_sc_params,
        out_type=[jax.ShapeDtypeStruct((E, D), jnp.float32),
                  jax.ShapeDtypeStruct((E, D), jnp.float32)],
        scratch_types=[pltpu.VMEM((EW,), jnp.int32),
                       pltpu.VMEM((EW,), jnp.int32),
                       pltpu.VMEM((B, D), jnp.float32),
                       pltpu.VMEM((B, D), jnp.float32),
                       pltpu.VMEM((B, D), jnp.float32),
                       pltpu.VMEM((B, D), jnp.float32),
                       pltpu.SemaphoreType.DMA,
                       pltpu.SemaphoreType.DMA,
                       pltpu.SemaphoreType.DMA,
                       pltpu.SemaphoreType.DMA],
    )
    def k(h1_hbm, ii_hbm, jj_hbm, hi_hbm, hj_hbm,
          ivf, jvf, ri0, rj0, ri1, rj1, s1, s2, s3, s4):
        wid = lax.axis_index("s") * NC + lax.axis_index("c")
        base = wid * EW
        pltpu.sync_copy(ii_hbm.at[pl.ds(base, EW)], ivf)
        pltpu.sync_copy(jj_hbm.at[pl.ds(base, EW)], jvf)

        def pair(bb, _):
            o0 = bb * 2 * B
            o1 = o0 + B
            c1 = pltpu.async_copy(h1_hbm.at[ivf.at[pl.ds(o0, B)]], ri0, s1)
            c2 = pltpu.async_copy(h1_hbm.at[jvf.at[pl.ds(o0, B)]], rj0, s2)
            c3 = pltpu.async_copy(h1_hbm.at[ivf.at[pl.ds(o1, B)]], ri1, s3)
            c4 = pltpu.async_copy(h1_hbm.at[jvf.at[pl.ds(o1, B)]], rj1, s4)
            c1.wait()
            c2.wait()
            w1 = pltpu.async_copy(ri0, hi_hbm.at[pl.ds(base + o0, B)], s1)
            w2 = pltpu.async_copy(rj0, hj_hbm.at[pl.ds(base + o0, B)], s2)
            c3.wait()
            c4.wait()
            w3 = pltpu.async_copy(ri1, hi_hbm.at[pl.ds(base + o1, B)], s3)
            w4 = pltpu.async_copy(rj1, hj_hbm.at[pl.ds(base + o1, B)], s4)
            w1.wait()
            w2.wait()
            w3.wait()
            w4.wait()
            return 0

        lax.fori_loop(0, NF // 2, pair, 0)
        o0 = (NF - 1) * B
        c1 = pltpu.async_copy(h1_hbm.at[ivf.at[pl.ds(o0, B)]], ri0, s1)
        c2 = pltpu.async_copy(h1_hbm.at[jvf.at[pl.ds(o0, B)]], rj0, s2)
        c3 = pltpu.async_copy(h1_hbm.at[ivf.at[pl.ds(NF * B, TL)]],
                              ri1.at[pl.ds(0, TL)], s3)
        c4 = pltpu.async_copy(h1_hbm.at[jvf.at[pl.ds(NF * B, TL)]],
                              rj1.at[pl.ds(0, TL)], s4)
        c1.wait()
        c2.wait()
        w1 = pltpu.async_copy(ri0, hi_hbm.at[pl.ds(base + o0, B)], s1)
        w2 = pltpu.async_copy(rj0, hj_hbm.at[pl.ds(base + o0, B)], s2)
        c3.wait()
        c4.wait()
        w3 = pltpu.async_copy(ri1.at[pl.ds(0, TL)],
                              hi_hbm.at[pl.ds(base + NF * B, TL)], s3)
        w4 = pltpu.async_copy(rj1.at[pl.ds(0, TL)],
                              hj_hbm.at[pl.ds(base + NF * B, TL)], s4)
        w1.wait()
        w2.wait()
        w3.wait()
        w4.wait()

    return k(h1, ii, jj)


def _sc_agg(mji, mkj, sb, jid, kid):
    C = 6400
    KC = E // C
    TS = T // NS
    NV = TS // 16
    B = 64
    MAXB = (TS + 127) // 128 + 1
    RW = C // NS

    @functools.partial(
        pl.kernel, mesh=_mesh(), compiler_params=_sc_params,
        out_type=jax.ShapeDtypeStruct((E, D), jnp.float32),
        scratch_types=[
            pltpu.VMEM((TS,), jnp.int32),
            pltpu.VMEM((TS,), jnp.int32),
            pltpu.VMEM((MAXB, 128), jnp.int32),
            pltpu.VMEM((MAXB, 128), jnp.int32),
            pltpu.VMEM((2, B), jnp.int32),
            pltpu.VMEM((2, B), jnp.int32),
            pltpu.VMEM((B, D), jnp.float32),
            pltpu.VMEM((B, D), jnp.float32),
            pltpu.VMEM((B, D), jnp.float32),
            pltpu.VMEM((B, D), jnp.float32),
            pltpu.VMEM_SHARED((C + 8, D), jnp.float32),
            pltpu.SemaphoreType.DMA,
            pltpu.SemaphoreType.DMA,
            pltpu.SemaphoreType.DMA,
            pltpu.SemaphoreType.DMA,
        ],
    )
    def k(mji_hbm, mkj_hbm, sb_hbm, jid_hbm, kid_hbm, out_hbm,
          jv_ref, kv_ref, cbk, cbtd, stg_t, stg_d, rowsA0, rowsB0,
          rowsA1, rowsB1, acc, s1, s2, s3, s4):
        cid = lax.axis_index("c")
        sid = lax.axis_index("s")
        wid = sid * NC + cid
        tbase = sid * TS
        pltpu.sync_copy(jid_hbm.at[pl.ds(tbase, TS)], jv_ref)
        pltpu.sync_copy(kid_hbm.at[pl.ds(tbase, TS)], kv_ref)
        lanes = lax.iota(jnp.int32, 16)
        ck_lo = cid * 13
        ck_hi = jnp.where(cid == 0, 13, KC)

        def unpack(slot, b):
            row = lax.shift_right_logical(b, 1)
            colb = lax.bitwise_and(b, 1) * B
            for u in range(B // 16):
                val = cbtd[row, pl.ds(colb + u * 16, 16)]
                stg_t[slot, pl.ds(u * 16, 16)] = (
                    tbase + lax.shift_right_logical(val, 13))
                stg_d[slot, pl.ds(u * 16, 16)] = lax.bitwise_and(val, 8191)

        def issue(slot, b, rowsA, rowsB, sa, sb_):
            unpack(slot, b)
            row = lax.shift_right_logical(b, 1)
            colb = lax.bitwise_and(b, 1) * B
            ca = pltpu.async_copy(
                mkj_hbm.at[cbk.at[row, pl.ds(colb, B)]], rowsA, sa)
            cb = pltpu.async_copy(sb_hbm.at[stg_t.at[slot]], rowsB, sb_)
            return ca, cb

        def mul_sc(slot, rowsA, rowsB):
            @pl.loop(0, B)
            def mul_row(r):
                for cc in range(0, D, 16):
                    rowsA[r, pl.ds(cc, 16)] = (rowsA[r, pl.ds(cc, 16)]
                                               * rowsB[r, pl.ds(cc, 16)])

            pltpu.sync_copy(rowsA, acc.at[stg_d.at[slot]], add=True)

        def chunk(ck, _):
            cbase = ck * C
            pltpu.sync_copy(mji_hbm.at[pl.ds(cbase + sid * RW, RW)],
                            acc.at[pl.ds(sid * RW, RW)])
            plsc.subcore_barrier()

            def scan_body(v, off):
                jx = jv_ref[pl.ds(v * 16, 16)]
                lj = jx - cbase
                msk = (lj >= 0) & (lj < C)
                ones = msk.astype(jnp.int32)
                tot = jnp.sum(ones)

                @pl.when(tot > 0)
                def store():
                    kx = kv_ref[pl.ds(v * 16, 16)]
                    inc = plsc.cumsum(ones)
                    pos = off + inc - 1
                    prow = lax.shift_right_logical(pos, 7)
                    pcol = lax.bitwise_and(pos, 127)
                    trel = v * 16 + lanes
                    packed = lax.bitwise_or(lax.shift_left(trel, 13), lj)
                    plsc.store_scatter(cbk, [prow, pcol], kx, mask=msk)
                    plsc.store_scatter(cbtd, [prow, pcol], packed, mask=msk)

                return off + tot

            off = lax.fori_loop(0, NV, scan_body, jnp.int32(0))
            nb = lax.shift_right_logical(off + B - 1, 6)
            pstart = lax.shift_right_logical(off, 4) * 16
            npv = lax.shift_right_logical(nb * B - pstart, 4)

            def pad_body(q, _):
                pos = pstart + q * 16 + lanes
                mskp = pos >= off
                prow = lax.shift_right_logical(pos, 7)
                pcol = lax.bitwise_and(pos, 127)
                padk = wid * 16 + lanes
                trel = lax.bitwise_and(lanes, 7)
                packed = lax.bitwise_or(lax.shift_left(trel, 13),
                                        C + trel)
                plsc.store_scatter(cbk, [prow, pcol], padk, mask=mskp)
                plsc.store_scatter(cbtd, [prow, pcol], packed, mask=mskp)
                return 0

            lax.fori_loop(0, npv, pad_body, 0)

            def pair_body(bb, _):
                b0 = bb * 2
                b1 = b0 + 1
                c1, c2 = issue(0, b0, rowsA0, rowsB0, s1, s2)
                c3, c4 = issue(1, b1, rowsA1, rowsB1, s3, s4)
                c1.wait()
                c2.wait()
                mul_sc(0, rowsA0, rowsB0)
                c3.wait()
                c4.wait()
                mul_sc(1, rowsA1, rowsB1)
                return 0

            lax.fori_loop(0, lax.shift_right_logical(nb, 1), pair_body, 0)

            @pl.when(lax.bitwise_and(nb, 1) == 1)
            def tail():
                c1, c2 = issue(0, nb - 1, rowsA0, rowsB0, s1, s2)
                c1.wait()
                c2.wait()
                mul_sc(0, rowsA0, rowsB0)

            plsc.subcore_barrier()
            pltpu.sync_copy(acc.at[pl.ds(sid * RW, RW)],
                            out_hbm.at[pl.ds(cbase + sid * RW, RW)])
            plsc.subcore_barrier()
            return 0

        lax.fori_loop(ck_lo, ck_hi, chunk, 0)

    return k(mji, mkj, sb, jid, kid)


def _sc_nagg(m2, ro, iidx):
    C0 = 5120
    ES = E // NS
    NV = ES // 16
    B = 64
    MAXB = (ES + 127) // 128 + 1
    ZB = 16

    @functools.partial(
        pl.kernel, mesh=_mesh(), compiler_params=_sc_params,
        out_type=jax.ShapeDtypeStruct((N, D), jnp.float32),
        scratch_types=[
            pltpu.VMEM((ES,), jnp.int32),
            pltpu.VMEM((MAXB, 128), jnp.int32),
            pltpu.VMEM((MAXB, 128), jnp.int32),
            pltpu.VMEM((2, B), jnp.int32),
            pltpu.VMEM((B, D), jnp.float32),
            pltpu.VMEM((B, D), jnp.float32),
            pltpu.VMEM((B, D), jnp.float32),
            pltpu.VMEM((B, D), jnp.float32),
            pltpu.VMEM((ZB, D), jnp.float32),
            pltpu.VMEM_SHARED((C0 + 8, D), jnp.float32),
            pltpu.SemaphoreType.DMA,
            pltpu.SemaphoreType.DMA,
            pltpu.SemaphoreType.DMA,
            pltpu.SemaphoreType.DMA,
        ],
    )
    def k(m2_hbm, ro_hbm, i_hbm, out_hbm,
          iv_ref, cbe, cbd, stg_d, rowsA0, rowsB0, rowsA1, rowsB1, zbuf,
          acc, s1, s2, s3, s4):
        cid = lax.axis_index("c")
        sid = lax.axis_index("s")
        wid = sid * NC + cid
        ebase = sid * ES
        pltpu.sync_copy(i_hbm.at[pl.ds(ebase, ES)], iv_ref)
        lanes = lax.iota(jnp.int32, 16)
        nbase = cid * C0
        climit = jnp.where(cid == 0, C0, N - C0)
        zoff = jnp.where(cid == 0, sid * 320, sid * 304)
        zn = jnp.where(cid == 0, 320,
                       jnp.where(sid < 15, 304, 320))

        @pl.loop(0, ZB)
        def zrow(r):
            for cc in range(0, D, 16):
                zbuf[r, pl.ds(cc, 16)] = jnp.zeros((16,), jnp.float32)

        def zcopy(z, _):
            pltpu.sync_copy(zbuf, acc.at[pl.ds(zoff + z * ZB, ZB)])
            return 0

        lax.fori_loop(0, zn // ZB, zcopy, 0)
        plsc.subcore_barrier()

        def scan_body(v, off):
            ix = iv_ref[pl.ds(v * 16, 16)]
            lj = ix - nbase
            msk = (lj >= 0) & (lj < climit)
            ones = msk.astype(jnp.int32)
            tot = jnp.sum(ones)

            @pl.when(tot > 0)
            def store():
                inc = plsc.cumsum(ones)
                pos = off + inc - 1
                prow = lax.shift_right_logical(pos, 7)
                pcol = lax.bitwise_and(pos, 127)
                evec = ebase + v * 16 + lanes
                plsc.store_scatter(cbe, [prow, pcol], evec, mask=msk)
                plsc.store_scatter(cbd, [prow, pcol], lj, mask=msk)

            return off + tot

        off = lax.fori_loop(0, NV, scan_body, jnp.int32(0))
        nb = lax.shift_right_logical(off + B - 1, 6)
        pstart = lax.shift_right_logical(off, 4) * 16
        npv = lax.shift_right_logical(nb * B - pstart, 4)

        def pad_body(q, _):
            pos = pstart + q * 16 + lanes
            mskp = pos >= off
            prow = lax.shift_right_logical(pos, 7)
            pcol = lax.bitwise_and(pos, 127)
            pade = wid * 16 + lanes
            padd = C0 + lax.bitwise_and(lanes, 7)
            plsc.store_scatter(cbe, [prow, pcol], pade, mask=mskp)
            plsc.store_scatter(cbd, [prow, pcol], padd, mask=mskp)
            return 0

        lax.fori_loop(0, npv, pad_body, 0)

        def stage(slot, b):
            row = lax.shift_right_logical(b, 1)
            colb = lax.bitwise_and(b, 1) * B
            for u in range(B // 16):
                stg_d[slot, pl.ds(u * 16, 16)] = cbd[row,
                                                     pl.ds(colb + u * 16, 16)]

        def issue(slot, b, rowsA, rowsB, sa, sb_):
            stage(slot, b)
            row = lax.shift_right_logical(b, 1)
            colb = lax.bitwise_and(b, 1) * B
            ca = pltpu.async_copy(
                m2_hbm.at[cbe.at[row, pl.ds(colb, B)]], rowsA, sa)
            cb = pltpu.async_copy(
                ro_hbm.at[cbe.at[row, pl.ds(colb, B)]], rowsB, sb_)
            return ca, cb

        def mul_sc(slot, rowsA, rowsB):
            @pl.loop(0, B)
            def mul_row(r):
                for cc in range(0, D, 16):
                    rowsA[r, pl.ds(cc, 16)] = (rowsA[r, pl.ds(cc, 16)]
                                               * rowsB[r, pl.ds(cc, 16)])

            pltpu.sync_copy(rowsA, acc.at[stg_d.at[slot]], add=True)

        def pair_body(bb, _):
            b0 = bb * 2
            b1 = b0 + 1
            c1, c2 = issue(0, b0, rowsA0, rowsB0, s1, s2)
            c3, c4 = issue(1, b1, rowsA1, rowsB1, s3, s4)
            c1.wait()
            c2.wait()
            mul_sc(0, rowsA0, rowsB0)
            c3.wait()
            c4.wait()
            mul_sc(1, rowsA1, rowsB1)
            return 0

        lax.fori_loop(0, lax.shift_right_logical(nb, 1), pair_body, 0)

        @pl.when(lax.bitwise_and(nb, 1) == 1)
        def tail():
            c1, c2 = issue(0, nb - 1, rowsA0, rowsB0, s1, s2)
            c1.wait()
            c2.wait()
            mul_sc(0, rowsA0, rowsB0)

        plsc.subcore_barrier()

        @pl.when(zn == 320)
        def wout():
            pltpu.sync_copy(acc.at[pl.ds(zoff, 320)],
                            out_hbm.at[pl.ds(nbase + zoff, 320)])

        @pl.when(zn == 304)
        def woutt():
            pltpu.sync_copy(acc.at[pl.ds(zoff, 304)],
                            out_hbm.at[pl.ds(nbase + zoff, 304)])

    return k(m2, ro, iidx)


def kernel(h, rbf, sbf1, sbf2, idx_kj, idx_ji_1, idx_jj, idx_ji_2,
           edge_index, params):
    p = params
    i32 = jnp.int32
    ii = edge_index[1].astype(i32)
    jj = edge_index[0].astype(i32)
    idx_kj = idx_kj.astype(i32)
    idx_ji_1 = idx_ji_1.astype(i32)
    idx_jj = idx_jj.astype(i32)
    idx_ji_2 = idx_ji_2.astype(i32)

    def b2(name):
        return p[name].reshape(1, D)

    kW, jW = p['kj_W'], p['ji1_W']

    h1 = _tc_node_mlp(h, p['h_W'], b2('h_b'))
    hi, hj = _sc_gather2(h1, ii, jj)
    sb1, sb2, ro = _tc_sbro(sbf1, sbf2, rbf,
                            p['s1a_W'], b2('s1a_b'), p['s1b_W'], b2('s1b_b'),
                            p['s2a_W'], b2('s2a_b'), p['s2b_W'], b2('s2b_b'),
                            p['rbfo_W'])
    m_kj, m_ji1 = _tc_edge1(hi, hj, rbf,
                            kW[:D], kW[D:2 * D], kW[2 * D:], b2('kj_b'),
                            p['rbf1_W'],
                            jW[:D], jW[D:2 * D], jW[2 * D:], b2('ji1_b'))
    m1 = _sc_agg(m_ji1, m_kj, sb1, idx_ji_1, idx_kj)
    m_jj, m_ji2 = _tc_edge2(m1, rbf, p['jj_W'], b2('jj_b'), p['rbf2_W'],
                            p['ji2_W'], b2('ji2_b'))
    m2 = _sc_agg(m_ji2, m_jj, sb2, idx_ji_2, idx_jj)
    h2raw = _sc_nagg(m2, ro, ii)

    yWpad = jnp.pad(p['yW_W'], ((0, 0), (0, D - 1)))
    ybpad = jnp.pad(p['yW_b'], (0, D - 1)).reshape(1, D)
    ws = [p['r1a_W'], b2('r1a_b'), p['r1b_W'], b2('r1b_b'),
          p['h_W'], b2('h_b'),
          p['r2a_W'], b2('r2a_b'), p['r2b_W'], b2('r2b_b'),
          p['r3a_W'], b2('r3a_b'), p['r3b_W'], b2('r3b_b'),
          p['y1_W'], b2('y1_b'), p['y2_W'], b2('y2_b'),
          p['y3_W'], b2('y3_b'), yWpad, ybpad]
    h2, yfull = _tc_final(h2raw, h, ws)
    return (h2, yfull[:, :1])


# speedup vs baseline: 1.7722x; 1.0297x over previous
import dataclasses
import functools

import jax
import jax.numpy as jnp
from jax import lax
from jax.experimental import pallas as pl
from jax.experimental.pallas import tpu as pltpu
from jax.experimental.pallas import tpu_sc as plsc

D = 128
N = 10000
E = 160000
T = 160000
NC = 2
NS = 16
NW = NC * NS

_mesh = functools.partial(plsc.VectorSubcoreMesh,
                          core_axis_name="c", subcore_axis_name="s")

_sc_params = pltpu.CompilerParams()
if "needs_layout_passes" in pltpu.CompilerParams.__dataclass_fields__:
    _sc_params = dataclasses.replace(_sc_params, needs_layout_passes=False)


def _silu(x):
    return x * jax.nn.sigmoid(x)


def _mm(x, w):
    return jax.lax.dot_general(x, w, (((1,), (0,)), ((), ())),
                               preferred_element_type=jnp.float32)


def _tc_node_mlp(h, W, b):
    BLK = 1000

    def body(h_ref, w_ref, b_ref, o_ref):
        o_ref[...] = _silu(_mm(h_ref[...], w_ref[...]) + b_ref[...])

    return pl.pallas_call(
        body,
        grid=(N // BLK,),
        in_specs=[pl.BlockSpec((BLK, D), lambda i: (i, 0)),
                  pl.BlockSpec((D, D), lambda i: (0, 0)),
                  pl.BlockSpec((1, D), lambda i: (0, 0))],
        out_specs=pl.BlockSpec((BLK, D), lambda i: (i, 0)),
        out_shape=jax.ShapeDtypeStruct((N, D), jnp.float32),
    )(h, W, b)


def _tc_sbro(sbf1, sbf2, rbf, s1a, s1ab, s1b, s1bb, s2a, s2ab, s2b, s2bb, ro_W):
    BLK = 2000

    def body(x1, x2, rb, wa1, ba1, wb1, bb1, wa2, ba2, wb2, bb2, wro,
             o1, o2, oro):
        o1[...] = _silu(_mm(_silu(_mm(x1[...], wa1[...]) + ba1[...]),
                            wb1[...]) + bb1[...])
        o2[...] = _silu(_mm(_silu(_mm(x2[...], wa2[...]) + ba2[...]),
                            wb2[...]) + bb2[...])
        oro[...] = _mm(rb[...], wro[...])

    full = pl.BlockSpec((D, D), lambda i: (0, 0))
    bias = pl.BlockSpec((1, D), lambda i: (0, 0))
    blk = pl.BlockSpec((BLK, D), lambda i: (i, 0))
    return pl.pallas_call(
        body,
        grid=(T // BLK,),
        in_specs=[blk, blk, blk, full, bias, full, bias, full, bias, full,
                  bias, full],
        out_specs=[blk, blk, blk],
        out_shape=[jax.ShapeDtypeStruct((T, D), jnp.float32),
                   jax.ShapeDtypeStruct((T, D), jnp.float32),
                   jax.ShapeDtypeStruct((E, D), jnp.float32)],
    )(sbf1, sbf2, rbf, s1a, s1ab, s1b, s1bb, s2a, s2ab, s2b, s2bb, ro_W)


def _tc_edge1(hi, hj, rbf, kA, kB, kC, kb, r1W, jA, jB, jC, jb):
    BLK = 2000

    def body(hi_r, hj_r, rb_r, kA_r, kB_r, kC_r, kb_r, r1_r, jA_r, jB_r,
             jC_r, jb_r, okj, oji):
        hi_x, hj_x, rb_x = hi_r[...], hj_r[...], rb_r[...]
        pre_k = (_mm(hi_x, kA_r[...]) + _mm(hj_x, kB_r[...])
                 + _mm(rb_x, kC_r[...]) + kb_r[...])
        okj[...] = _silu(pre_k) * _mm(rb_x, r1_r[...])
        pre_j = (_mm(hi_x, jA_r[...]) + _mm(hj_x, jB_r[...])
                 + _mm(rb_x, jC_r[...]) + jb_r[...])
        oji[...] = _silu(pre_j)

    full = pl.BlockSpec((D, D), lambda i: (0, 0))
    bias = pl.BlockSpec((1, D), lambda i: (0, 0))
    blk = pl.BlockSpec((BLK, D), lambda i: (i, 0))
    return pl.pallas_call(
        body,
        grid=(E // BLK,),
        in_specs=[blk, blk, blk, full, full, full, bias, full, full, full,
                  full, bias],
        out_specs=[blk, blk],
        out_shape=[jax.ShapeDtypeStruct((E, D), jnp.float32),
                   jax.ShapeDtypeStruct((E, D), jnp.float32)],
    )(hi, hj, rbf, kA, kB, kC, kb, r1W, jA, jB, jC, jb)


def _tc_edge2(m1, rbf, jjW, jjb, r2W, j2W, j2b):
    BLK = 2000

    def body(m_r, rb_r, jjW_r, jjb_r, r2_r, j2W_r, j2b_r, ojj, oji):
        m_x, rb_x = m_r[...], rb_r[...]
        ojj[...] = _silu(_mm(m_x, jjW_r[...]) + jjb_r[...]) * _mm(rb_x, r2_r[...])
        oji[...] = _silu(_mm(m_x, j2W_r[...]) + j2b_r[...])

    full = pl.BlockSpec((D, D), lambda i: (0, 0))
    bias = pl.BlockSpec((1, D), lambda i: (0, 0))
    blk = pl.BlockSpec((BLK, D), lambda i: (i, 0))
    return pl.pallas_call(
        body,
        grid=(E // BLK,),
        in_specs=[blk, blk, full, bias, full, full, bias],
        out_specs=[blk, blk],
        out_shape=[jax.ShapeDtypeStruct((E, D), jnp.float32),
                   jax.ShapeDtypeStruct((E, D), jnp.float32)],
    )(m1, rbf, jjW, jjb, r2W, j2W, j2b)


def _tc_final(praw, h, ws):
    BLK = 1000

    def body(p_r, h_r,
             r1a, r1ab, r1b, r1bb, hW, hb, r2a, r2ab, r2b, r2bb,
             r3a, r3ab, r3b, r3bb, y1, y1b, y2, y2b, y3, y3b, yW, ybp,
             oh, oy):
        x = p_r[...]

        def res(x, wa, ba, wb, bb):
            return _silu(_mm(_silu(_mm(x, wa[...]) + ba[...]), wb[...])
                         + bb[...]) + x

        x = res(x, r1a, r1ab, r1b, r1bb)
        x = _silu(_mm(x, hW[...]) + hb[...]) + h_r[...]
        x = res(x, r2a, r2ab, r2b, r2bb)
        x = res(x, r3a, r3ab, r3b, r3bb)
        oh[...] = x
        t = _silu(_mm(x, y1[...]) + y1b[...])
        t = _silu(_mm(t, y2[...]) + y2b[...])
        t = _silu(_mm(t, y3[...]) + y3b[...])
        oy[...] = _mm(t, yW[...]) + ybp[...]

    full = pl.BlockSpec((D, D), lambda i: (0, 0))
    bias = pl.BlockSpec((1, D), lambda i: (0, 0))
    blk = pl.BlockSpec((BLK, D), lambda i: (i, 0))
    return pl.pallas_call(
        body,
        grid=(N // BLK,),
        in_specs=[blk, blk] + [full, bias] * 11,
        out_specs=[blk, blk],
        out_shape=[jax.ShapeDtypeStruct((N, D), jnp.float32),
                   jax.ShapeDtypeStruct((N, D), jnp.float32)],
    )(praw, h, *ws)


def _sc_gather2(h1, ii, jj):
    B = 64
    EW = E // NW
    NF = EW // B
    TL = EW - NF * B

    @functools.partial(
        pl.kernel, mesh=_mesh(), compiler_params=_sc_params,
        out_type=[jax.ShapeDtypeStruct((E, D), jnp.float32),
                  jax.ShapeDtypeStruct((E, D), jnp.float32)],
        scratch_types=[pltpu.VMEM((EW,), jnp.int32),
                       pltpu.VMEM((EW,), jnp.int32),
                       pltpu.VMEM((B, D), jnp.float32),
                       pltpu.VMEM((B, D), jnp.float32),
                       pltpu.VMEM((B, D), jnp.float32),
                       pltpu.VMEM((B, D), jnp.float32),
                       pltpu.VMEM_SHARED((N, D), jnp.float32),
                       pltpu.SemaphoreType.DMA,
                       pltpu.SemaphoreType.DMA,
                       pltpu.SemaphoreType.DMA,
                       pltpu.SemaphoreType.DMA],
    )
    def k(h1_hbm, ii_hbm, jj_hbm, hi_hbm, hj_hbm,
          ivf, jvf, ri0, rj0, ri1, rj1, table, s1, s2, s3, s4):
        cid = lax.axis_index("c")
        sid = lax.axis_index("s")
        wid = sid * NC + cid
        base = wid * EW
        toff = sid * 640
        tn_full = jnp.where(sid < 15, 1, 0)

        @pl.when(sid < 15)
        def tl_a():
            pltpu.sync_copy(h1_hbm.at[pl.ds(toff, 640)],
                            table.at[pl.ds(toff, 640)])

        @pl.when(sid == 15)
        def tl_b():
            pltpu.sync_copy(h1_hbm.at[pl.ds(9600, 400)],
                            table.at[pl.ds(9600, 400)])

        pltpu.sync_copy(ii_hbm.at[pl.ds(base, EW)], ivf)
        pltpu.sync_copy(jj_hbm.at[pl.ds(base, EW)], jvf)
        plsc.subcore_barrier()

        def pair(bb, _):
            o0 = bb * 2 * B
            o1 = o0 + B
            c1 = pltpu.async_copy(table.at[ivf.at[pl.ds(o0, B)]], ri0, s1)
            c2 = pltpu.async_copy(table.at[jvf.at[pl.ds(o0, B)]], rj0, s2)
            c3 = pltpu.async_copy(table.at[ivf.at[pl.ds(o1, B)]], ri1, s3)
            c4 = pltpu.async_copy(table.at[jvf.at[pl.ds(o1, B)]], rj1, s4)
            c1.wait()
            c2.wait()
            w1 = pltpu.async_copy(ri0, hi_hbm.at[pl.ds(base + o0, B)], s1)
            w2 = pltpu.async_copy(rj0, hj_hbm.at[pl.ds(base + o0, B)], s2)
            c3.wait()
            c4.wait()
            w3 = pltpu.async_copy(ri1, hi_hbm.at[pl.ds(base + o1, B)], s3)
            w4 = pltpu.async_copy(rj1, hj_hbm.at[pl.ds(base + o1, B)], s4)
            w1.wait()
            w2.wait()
            w3.wait()
            w4.wait()
            return 0

        lax.fori_loop(0, NF // 2, pair, 0)
        c3 = pltpu.async_copy(table.at[ivf.at[pl.ds(NF * B, TL)]],
                              ri1.at[pl.ds(0, TL)], s3)
        c4 = pltpu.async_copy(table.at[jvf.at[pl.ds(NF * B, TL)]],
                              rj1.at[pl.ds(0, TL)], s4)
        c3.wait()
        c4.wait()
        w3 = pltpu.async_copy(ri1.at[pl.ds(0, TL)],
                              hi_hbm.at[pl.ds(base + NF * B, TL)], s3)
        w4 = pltpu.async_copy(rj1.at[pl.ds(0, TL)],
                              hj_hbm.at[pl.ds(base + NF * B, TL)], s4)
        w3.wait()
        w4.wait()

    return k(h1, ii, jj)


def _sc_agg(mji, mkj, sb, jid, kid):
    C = 6400
    KC = E // C
    TS = T // NS
    NV = TS // 16
    B = 64
    MAXB = (TS + 127) // 128 + 1
    RW = C // NS

    @functools.partial(
        pl.kernel, mesh=_mesh(), compiler_params=_sc_params,
        out_type=jax.ShapeDtypeStruct((E, D), jnp.float32),
        scratch_types=[
            pltpu.VMEM((TS,), jnp.int32),
            pltpu.VMEM((TS,), jnp.int32),
            pltpu.VMEM((MAXB, 128), jnp.int32),
            pltpu.VMEM((MAXB, 128), jnp.int32),
            pltpu.VMEM((2, B), jnp.int32),
            pltpu.VMEM((2, B), jnp.int32),
            pltpu.VMEM((B, D), jnp.float32),
            pltpu.VMEM((B, D), jnp.float32),
            pltpu.VMEM((B, D), jnp.float32),
            pltpu.VMEM((B, D), jnp.float32),
            pltpu.VMEM_SHARED((C + 8, D), jnp.float32),
            pltpu.SemaphoreType.DMA,
            pltpu.SemaphoreType.DMA,
            pltpu.SemaphoreType.DMA,
            pltpu.SemaphoreType.DMA,
        ],
    )
    def k(mji_hbm, mkj_hbm, sb_hbm, jid_hbm, kid_hbm, out_hbm,
          jv_ref, kv_ref, cbk, cbtd, stg_t, stg_d, rowsA0, rowsB0,
          rowsA1, rowsB1, acc, s1, s2, s3, s4):
        cid = lax.axis_index("c")
        sid = lax.axis_index("s")
        wid = sid * NC + cid
        tbase = sid * TS
        pltpu.sync_copy(jid_hbm.at[pl.ds(tbase, TS)], jv_ref)
        pltpu.sync_copy(kid_hbm.at[pl.ds(tbase, TS)], kv_ref)
        lanes = lax.iota(jnp.int32, 16)
        ck_lo = cid * 13
        ck_hi = jnp.where(cid == 0, 13, KC)

        def unpack(slot, b):
            row = lax.shift_right_logical(b, 1)
            colb = lax.bitwise_and(b, 1) * B
            for u in range(B // 16):
                val = cbtd[row, pl.ds(colb + u * 16, 16)]
                stg_t[slot, pl.ds(u * 16, 16)] = (
                    tbase + lax.shift_right_logical(val, 13))
                stg_d[slot, pl.ds(u * 16, 16)] = lax.bitwise_and(val, 8191)

        def issue(slot, b, rowsA, rowsB, sa, sb_):
            unpack(slot, b)
            row = lax.shift_right_logical(b, 1)
            colb = lax.bitwise_and(b, 1) * B
            ca = pltpu.async_copy(
                mkj_hbm.at[cbk.at[row, pl.ds(colb, B)]], rowsA, sa)
            cb = pltpu.async_copy(sb_hbm.at[stg_t.at[slot]], rowsB, sb_)
            return ca, cb

        def mul_sc(slot, rowsA, rowsB):
            @pl.loop(0, B)
            def mul_row(r):
                for cc in range(0, D, 16):
                    rowsA[r, pl.ds(cc, 16)] = (rowsA[r, pl.ds(cc, 16)]
                                               * rowsB[r, pl.ds(cc, 16)])

            pltpu.sync_copy(rowsA, acc.at[stg_d.at[slot]], add=True)

        def chunk(ck, _):
            cbase = ck * C
            pltpu.sync_copy(mji_hbm.at[pl.ds(cbase + sid * RW, RW)],
                            acc.at[pl.ds(sid * RW, RW)])
            plsc.subcore_barrier()

            def scan_body(v, off):
                jx = jv_ref[pl.ds(v * 16, 16)]
                lj = jx - cbase
                msk = (lj >= 0) & (lj < C)
                ones = msk.astype(jnp.int32)
                tot = jnp.sum(ones)

                @pl.when(tot > 0)
                def store():
                    kx = kv_ref[pl.ds(v * 16, 16)]
                    inc = plsc.cumsum(ones)
                    pos = off + inc - 1
                    prow = lax.shift_right_logical(pos, 7)
                    pcol = lax.bitwise_and(pos, 127)
                    trel = v * 16 + lanes
                    packed = lax.bitwise_or(lax.shift_left(trel, 13), lj)
                    plsc.store_scatter(cbk, [prow, pcol], kx, mask=msk)
                    plsc.store_scatter(cbtd, [prow, pcol], packed, mask=msk)

                return off + tot

            off = lax.fori_loop(0, NV, scan_body, jnp.int32(0))
            nb = lax.shift_right_logical(off + B - 1, 6)
            pstart = lax.shift_right_logical(off, 4) * 16
            npv = lax.shift_right_logical(nb * B - pstart, 4)

            def pad_body(q, _):
                pos = pstart + q * 16 + lanes
                mskp = pos >= off
                prow = lax.shift_right_logical(pos, 7)
                pcol = lax.bitwise_and(pos, 127)
                padk = wid * 16 + lanes
                trel = lax.bitwise_and(lanes, 7)
                packed = lax.bitwise_or(lax.shift_left(trel, 13),
                                        C + trel)
                plsc.store_scatter(cbk, [prow, pcol], padk, mask=mskp)
                plsc.store_scatter(cbtd, [prow, pcol], packed, mask=mskp)
                return 0

            lax.fori_loop(0, npv, pad_body, 0)

            def pair_body(bb, _):
                b0 = bb * 2
                b1 = b0 + 1
                c1, c2 = issue(0, b0, rowsA0, rowsB0, s1, s2)
                c3, c4 = issue(1, b1, rowsA1, rowsB1, s3, s4)
                c1.wait()
                c2.wait()
                mul_sc(0, rowsA0, rowsB0)
                c3.wait()
                c4.wait()
                mul_sc(1, rowsA1, rowsB1)
                return 0

            lax.fori_loop(0, lax.shift_right_logical(nb, 1), pair_body, 0)

            @pl.when(lax.bitwise_and(nb, 1) == 1)
            def tail():
                c1, c2 = issue(0, nb - 1, rowsA0, rowsB0, s1, s2)
                c1.wait()
                c2.wait()
                mul_sc(0, rowsA0, rowsB0)

            plsc.subcore_barrier()
            pltpu.sync_copy(acc.at[pl.ds(sid * RW, RW)],
                            out_hbm.at[pl.ds(cbase + sid * RW, RW)])
            plsc.subcore_barrier()
            return 0

        lax.fori_loop(ck_lo, ck_hi, chunk, 0)

    return k(mji, mkj, sb, jid, kid)


def _sc_nagg(m2, ro, iidx):
    C0 = 5120
    ES = E // NS
    NV = ES // 16
    B = 64
    MAXB = (ES + 127) // 128 + 1
    ZB = 16

    @functools.partial(
        pl.kernel, mesh=_mesh(), compiler_params=_sc_params,
        out_type=jax.ShapeDtypeStruct((N, D), jnp.float32),
        scratch_types=[
            pltpu.VMEM((ES,), jnp.int32),
            pltpu.VMEM((MAXB, 128), jnp.int32),
            pltpu.VMEM((MAXB, 128), jnp.int32),
            pltpu.VMEM((2, B), jnp.int32),
            pltpu.VMEM((B, D), jnp.float32),
            pltpu.VMEM((B, D), jnp.float32),
            pltpu.VMEM((B, D), jnp.float32),
            pltpu.VMEM((B, D), jnp.float32),
            pltpu.VMEM((ZB, D), jnp.float32),
            pltpu.VMEM_SHARED((C0 + 8, D), jnp.float32),
            pltpu.SemaphoreType.DMA,
            pltpu.SemaphoreType.DMA,
            pltpu.SemaphoreType.DMA,
            pltpu.SemaphoreType.DMA,
        ],
    )
    def k(m2_hbm, ro_hbm, i_hbm, out_hbm,
          iv_ref, cbe, cbd, stg_d, rowsA0, rowsB0, rowsA1, rowsB1, zbuf,
          acc, s1, s2, s3, s4):
        cid = lax.axis_index("c")
        sid = lax.axis_index("s")
        wid = sid * NC + cid
        ebase = sid * ES
        pltpu.sync_copy(i_hbm.at[pl.ds(ebase, ES)], iv_ref)
        lanes = lax.iota(jnp.int32, 16)
        nbase = cid * C0
        climit = jnp.where(cid == 0, C0, N - C0)
        zoff = jnp.where(cid == 0, sid * 320, sid * 304)
        zn = jnp.where(cid == 0, 320,
                       jnp.where(sid < 15, 304, 320))

        @pl.loop(0, ZB)
        def zrow(r):
            for cc in range(0, D, 16):
                zbuf[r, pl.ds(cc, 16)] = jnp.zeros((16,), jnp.float32)

        def zcopy(z, _):
            pltpu.sync_copy(zbuf, acc.at[pl.ds(zoff + z * ZB, ZB)])
            return 0

        lax.fori_loop(0, zn // ZB, zcopy, 0)
        plsc.subcore_barrier()

        def scan_body(v, off):
            ix = iv_ref[pl.ds(v * 16, 16)]
            lj = ix - nbase
            msk = (lj >= 0) & (lj < climit)
            ones = msk.astype(jnp.int32)
            tot = jnp.sum(ones)

            @pl.when(tot > 0)
            def store():
                inc = plsc.cumsum(ones)
                pos = off + inc - 1
                prow = lax.shift_right_logical(pos, 7)
                pcol = lax.bitwise_and(pos, 127)
                evec = ebase + v * 16 + lanes
                plsc.store_scatter(cbe, [prow, pcol], evec, mask=msk)
                plsc.store_scatter(cbd, [prow, pcol], lj, mask=msk)

            return off + tot

        off = lax.fori_loop(0, NV, scan_body, jnp.int32(0))
        nb = lax.shift_right_logical(off + B - 1, 6)
        pstart = lax.shift_right_logical(off, 4) * 16
        npv = lax.shift_right_logical(nb * B - pstart, 4)

        def pad_body(q, _):
            pos = pstart + q * 16 + lanes
            mskp = pos >= off
            prow = lax.shift_right_logical(pos, 7)
            pcol = lax.bitwise_and(pos, 127)
            pade = wid * 16 + lanes
            padd = C0 + lax.bitwise_and(lanes, 7)
            plsc.store_scatter(cbe, [prow, pcol], pade, mask=mskp)
            plsc.store_scatter(cbd, [prow, pcol], padd, mask=mskp)
            return 0

        lax.fori_loop(0, npv, pad_body, 0)

        def stage(slot, b):
            row = lax.shift_right_logical(b, 1)
            colb = lax.bitwise_and(b, 1) * B
            for u in range(B // 16):
                stg_d[slot, pl.ds(u * 16, 16)] = cbd[row,
                                                     pl.ds(colb + u * 16, 16)]

        def issue(slot, b, rowsA, rowsB, sa, sb_):
            stage(slot, b)
            row = lax.shift_right_logical(b, 1)
            colb = lax.bitwise_and(b, 1) * B
            ca = pltpu.async_copy(
                m2_hbm.at[cbe.at[row, pl.ds(colb, B)]], rowsA, sa)
            cb = pltpu.async_copy(
                ro_hbm.at[cbe.at[row, pl.ds(colb, B)]], rowsB, sb_)
            return ca, cb

        def mul_sc(slot, rowsA, rowsB):
            @pl.loop(0, B)
            def mul_row(r):
                for cc in range(0, D, 16):
                    rowsA[r, pl.ds(cc, 16)] = (rowsA[r, pl.ds(cc, 16)]
                                               * rowsB[r, pl.ds(cc, 16)])

            pltpu.sync_copy(rowsA, acc.at[stg_d.at[slot]], add=True)

        def pair_body(bb, _):
            b0 = bb * 2
            b1 = b0 + 1
            c1, c2 = issue(0, b0, rowsA0, rowsB0, s1, s2)
            c3, c4 = issue(1, b1, rowsA1, rowsB1, s3, s4)
            c1.wait()
            c2.wait()
            mul_sc(0, rowsA0, rowsB0)
            c3.wait()
            c4.wait()
            mul_sc(1, rowsA1, rowsB1)
            return 0

        lax.fori_loop(0, lax.shift_right_logical(nb, 1), pair_body, 0)

        @pl.when(lax.bitwise_and(nb, 1) == 1)
        def tail():
            c1, c2 = issue(0, nb - 1, rowsA0, rowsB0, s1, s2)
            c1.wait()
            c2.wait()
            mul_sc(0, rowsA0, rowsB0)

        plsc.subcore_barrier()

        @pl.when(zn == 320)
        def wout():
            pltpu.sync_copy(acc.at[pl.ds(zoff, 320)],
                            out_hbm.at[pl.ds(nbase + zoff, 320)])

        @pl.when(zn == 304)
        def woutt():
            pltpu.sync_copy(acc.at[pl.ds(zoff, 304)],
                            out_hbm.at[pl.ds(nbase + zoff, 304)])

    return k(m2, ro, iidx)


def kernel(h, rbf, sbf1, sbf2, idx_kj, idx_ji_1, idx_jj, idx_ji_2,
           edge_index, params):
    p = params
    i32 = jnp.int32
    ii = edge_index[1].astype(i32)
    jj = edge_index[0].astype(i32)
    idx_kj = idx_kj.astype(i32)
    idx_ji_1 = idx_ji_1.astype(i32)
    idx_jj = idx_jj.astype(i32)
    idx_ji_2 = idx_ji_2.astype(i32)

    def b2(name):
        return p[name].reshape(1, D)

    kW, jW = p['kj_W'], p['ji1_W']

    h1 = _tc_node_mlp(h, p['h_W'], b2('h_b'))
    hi, hj = _sc_gather2(h1, ii, jj)
    sb1, sb2, ro = _tc_sbro(sbf1, sbf2, rbf,
                            p['s1a_W'], b2('s1a_b'), p['s1b_W'], b2('s1b_b'),
                            p['s2a_W'], b2('s2a_b'), p['s2b_W'], b2('s2b_b'),
                            p['rbfo_W'])
    m_kj, m_ji1 = _tc_edge1(hi, hj, rbf,
                            kW[:D], kW[D:2 * D], kW[2 * D:], b2('kj_b'),
                            p['rbf1_W'],
                            jW[:D], jW[D:2 * D], jW[2 * D:], b2('ji1_b'))
    m1 = _sc_agg(m_ji1, m_kj, sb1, idx_ji_1, idx_kj)
    m_jj, m_ji2 = _tc_edge2(m1, rbf, p['jj_W'], b2('jj_b'), p['rbf2_W'],
                            p['ji2_W'], b2('ji2_b'))
    m2 = _sc_agg(m_ji2, m_jj, sb2, idx_ji_2, idx_jj)
    h2raw = _sc_nagg(m2, ro, ii)

    yWpad = jnp.pad(p['yW_W'], ((0, 0), (0, D - 1)))
    ybpad = jnp.pad(p['yW_b'], (0, D - 1)).reshape(1, D)
    ws = [p['r1a_W'], b2('r1a_b'), p['r1b_W'], b2('r1b_b'),
          p['h_W'], b2('h_b'),
          p['r2a_W'], b2('r2a_b'), p['r2b_W'], b2('r2b_b'),
          p['r3a_W'], b2('r3a_b'), p['r3b_W'], b2('r3b_b'),
          p['y1_W'], b2('y1_b'), p['y2_W'], b2('y2_b'),
          p['y3_W'], b2('y3_b'), yWpad, ybpad]
    h2, yfull = _tc_final(h2raw, h, ws)
    return (h2, yfull[:, :1])


# speedup vs baseline: 1.9443x; 1.0971x over previous
import dataclasses
import functools

import jax
import jax.numpy as jnp
from jax import lax
from jax.experimental import pallas as pl
from jax.experimental.pallas import tpu as pltpu
from jax.experimental.pallas import tpu_sc as plsc

D = 128
N = 10000
E = 160000
T = 160000
NC = 2
NS = 16
NW = NC * NS

_mesh = functools.partial(plsc.VectorSubcoreMesh,
                          core_axis_name="c", subcore_axis_name="s")

_sc_params = pltpu.CompilerParams()
if "needs_layout_passes" in pltpu.CompilerParams.__dataclass_fields__:
    _sc_params = dataclasses.replace(_sc_params, needs_layout_passes=False)


def _silu(x):
    return x * jax.nn.sigmoid(x)


def _mm(x, w):
    return jax.lax.dot_general(x, w, (((1,), (0,)), ((), ())),
                               preferred_element_type=jnp.float32)


def _tc_node_mlp(h, W, b):
    BLK = 1000

    def body(h_ref, w_ref, b_ref, o_ref):
        o_ref[...] = _silu(_mm(h_ref[...], w_ref[...]) + b_ref[...])

    return pl.pallas_call(
        body,
        grid=(N // BLK,),
        in_specs=[pl.BlockSpec((BLK, D), lambda i: (i, 0)),
                  pl.BlockSpec((D, D), lambda i: (0, 0)),
                  pl.BlockSpec((1, D), lambda i: (0, 0))],
        out_specs=pl.BlockSpec((BLK, D), lambda i: (i, 0)),
        out_shape=jax.ShapeDtypeStruct((N, D), jnp.float32),
    )(h, W, b)


def _tc_sbro(sbf1, sbf2, rbf, s1a, s1ab, s1b, s1bb, s2a, s2ab, s2b, s2bb, ro_W):
    BLK = 2000

    def body(x1, x2, rb, wa1, ba1, wb1, bb1, wa2, ba2, wb2, bb2, wro,
             o1, o2, oro):
        o1[...] = _silu(_mm(_silu(_mm(x1[...], wa1[...]) + ba1[...]),
                            wb1[...]) + bb1[...])
        o2[...] = _silu(_mm(_silu(_mm(x2[...], wa2[...]) + ba2[...]),
                            wb2[...]) + bb2[...])
        oro[...] = _mm(rb[...], wro[...])

    full = pl.BlockSpec((D, D), lambda i: (0, 0))
    bias = pl.BlockSpec((1, D), lambda i: (0, 0))
    blk = pl.BlockSpec((BLK, D), lambda i: (i, 0))
    return pl.pallas_call(
        body,
        grid=(T // BLK,),
        in_specs=[blk, blk, blk, full, bias, full, bias, full, bias, full,
                  bias, full],
        out_specs=[blk, blk, blk],
        out_shape=[jax.ShapeDtypeStruct((T, D), jnp.float32),
                   jax.ShapeDtypeStruct((T, D), jnp.float32),
                   jax.ShapeDtypeStruct((E, D), jnp.float32)],
    )(sbf1, sbf2, rbf, s1a, s1ab, s1b, s1bb, s2a, s2ab, s2b, s2bb, ro_W)


def _tc_edge1(hi, hj, rbf, kA, kB, kC, kb, r1W, jA, jB, jC, jb):
    BLK = 2000

    def body(hi_r, hj_r, rb_r, kA_r, kB_r, kC_r, kb_r, r1_r, jA_r, jB_r,
             jC_r, jb_r, okj, oji):
        hi_x, hj_x, rb_x = hi_r[...], hj_r[...], rb_r[...]
        pre_k = (_mm(hi_x, kA_r[...]) + _mm(hj_x, kB_r[...])
                 + _mm(rb_x, kC_r[...]) + kb_r[...])
        okj[...] = _silu(pre_k) * _mm(rb_x, r1_r[...])
        pre_j = (_mm(hi_x, jA_r[...]) + _mm(hj_x, jB_r[...])
                 + _mm(rb_x, jC_r[...]) + jb_r[...])
        oji[...] = _silu(pre_j)

    full = pl.BlockSpec((D, D), lambda i: (0, 0))
    bias = pl.BlockSpec((1, D), lambda i: (0, 0))
    blk = pl.BlockSpec((BLK, D), lambda i: (i, 0))
    return pl.pallas_call(
        body,
        grid=(E // BLK,),
        in_specs=[blk, blk, blk, full, full, full, bias, full, full, full,
                  full, bias],
        out_specs=[blk, blk],
        out_shape=[jax.ShapeDtypeStruct((E, D), jnp.float32),
                   jax.ShapeDtypeStruct((E, D), jnp.float32)],
    )(hi, hj, rbf, kA, kB, kC, kb, r1W, jA, jB, jC, jb)


def _tc_edge2(m1, rbf, jjW, jjb, r2W, j2W, j2b):
    BLK = 2000

    def body(m_r, rb_r, jjW_r, jjb_r, r2_r, j2W_r, j2b_r, ojj, oji):
        m_x, rb_x = m_r[...], rb_r[...]
        ojj[...] = _silu(_mm(m_x, jjW_r[...]) + jjb_r[...]) * _mm(rb_x, r2_r[...])
        oji[...] = _silu(_mm(m_x, j2W_r[...]) + j2b_r[...])

    full = pl.BlockSpec((D, D), lambda i: (0, 0))
    bias = pl.BlockSpec((1, D), lambda i: (0, 0))
    blk = pl.BlockSpec((BLK, D), lambda i: (i, 0))
    return pl.pallas_call(
        body,
        grid=(E // BLK,),
        in_specs=[blk, blk, full, bias, full, full, bias],
        out_specs=[blk, blk],
        out_shape=[jax.ShapeDtypeStruct((E, D), jnp.float32),
                   jax.ShapeDtypeStruct((E, D), jnp.float32)],
    )(m1, rbf, jjW, jjb, r2W, j2W, j2b)


def _tc_final(praw, h, ws):
    BLK = 1000

    def body(p_r, h_r,
             r1a, r1ab, r1b, r1bb, hW, hb, r2a, r2ab, r2b, r2bb,
             r3a, r3ab, r3b, r3bb, y1, y1b, y2, y2b, y3, y3b, yW, ybp,
             oh, oy):
        x = p_r[...]

        def res(x, wa, ba, wb, bb):
            return _silu(_mm(_silu(_mm(x, wa[...]) + ba[...]), wb[...])
                         + bb[...]) + x

        x = res(x, r1a, r1ab, r1b, r1bb)
        x = _silu(_mm(x, hW[...]) + hb[...]) + h_r[...]
        x = res(x, r2a, r2ab, r2b, r2bb)
        x = res(x, r3a, r3ab, r3b, r3bb)
        oh[...] = x
        t = _silu(_mm(x, y1[...]) + y1b[...])
        t = _silu(_mm(t, y2[...]) + y2b[...])
        t = _silu(_mm(t, y3[...]) + y3b[...])
        oy[...] = _mm(t, yW[...]) + ybp[...]

    full = pl.BlockSpec((D, D), lambda i: (0, 0))
    bias = pl.BlockSpec((1, D), lambda i: (0, 0))
    blk = pl.BlockSpec((BLK, D), lambda i: (i, 0))
    return pl.pallas_call(
        body,
        grid=(N // BLK,),
        in_specs=[blk, blk] + [full, bias] * 11,
        out_specs=[blk, blk],
        out_shape=[jax.ShapeDtypeStruct((N, D), jnp.float32),
                   jax.ShapeDtypeStruct((N, D), jnp.float32)],
    )(praw, h, *ws)


def _sc_gather2(h1, ii, jj):
    B = 64
    EW = E // NW
    NF = EW // B
    TL = EW - NF * B

    @functools.partial(
        pl.kernel, mesh=_mesh(), compiler_params=_sc_params,
        out_type=[jax.ShapeDtypeStruct((E, D), jnp.float32),
                  jax.ShapeDtypeStruct((E, D), jnp.float32)],
        scratch_types=[pltpu.VMEM((EW,), jnp.int32),
                       pltpu.VMEM((EW,), jnp.int32),
                       pltpu.VMEM((B, D), jnp.float32),
                       pltpu.VMEM((B, D), jnp.float32),
                       pltpu.VMEM((B, D), jnp.float32),
                       pltpu.VMEM((B, D), jnp.float32),
                       pltpu.VMEM_SHARED((N, D), jnp.float32),
                       pltpu.SemaphoreType.DMA,
                       pltpu.SemaphoreType.DMA,
                       pltpu.SemaphoreType.DMA,
                       pltpu.SemaphoreType.DMA],
    )
    def k(h1_hbm, ii_hbm, jj_hbm, hi_hbm, hj_hbm,
          ivf, jvf, ri0, rj0, ri1, rj1, table, s1, s2, s3, s4):
        cid = lax.axis_index("c")
        sid = lax.axis_index("s")
        wid = sid * NC + cid
        base = wid * EW
        toff = sid * 640
        tn_full = jnp.where(sid < 15, 1, 0)

        @pl.when(sid < 15)
        def tl_a():
            pltpu.sync_copy(h1_hbm.at[pl.ds(toff, 640)],
                            table.at[pl.ds(toff, 640)])

        @pl.when(sid == 15)
        def tl_b():
            pltpu.sync_copy(h1_hbm.at[pl.ds(9600, 400)],
                            table.at[pl.ds(9600, 400)])

        pltpu.sync_copy(ii_hbm.at[pl.ds(base, EW)], ivf)
        pltpu.sync_copy(jj_hbm.at[pl.ds(base, EW)], jvf)
        plsc.subcore_barrier()

        def pair(bb, _):
            o0 = bb * 2 * B
            o1 = o0 + B
            c1 = pltpu.async_copy(table.at[ivf.at[pl.ds(o0, B)]], ri0, s1)
            c2 = pltpu.async_copy(table.at[jvf.at[pl.ds(o0, B)]], rj0, s2)
            c3 = pltpu.async_copy(table.at[ivf.at[pl.ds(o1, B)]], ri1, s3)
            c4 = pltpu.async_copy(table.at[jvf.at[pl.ds(o1, B)]], rj1, s4)
            c1.wait()
            c2.wait()
            w1 = pltpu.async_copy(ri0, hi_hbm.at[pl.ds(base + o0, B)], s1)
            w2 = pltpu.async_copy(rj0, hj_hbm.at[pl.ds(base + o0, B)], s2)
            c3.wait()
            c4.wait()
            w3 = pltpu.async_copy(ri1, hi_hbm.at[pl.ds(base + o1, B)], s3)
            w4 = pltpu.async_copy(rj1, hj_hbm.at[pl.ds(base + o1, B)], s4)
            w1.wait()
            w2.wait()
            w3.wait()
            w4.wait()
            return 0

        lax.fori_loop(0, NF // 2, pair, 0)
        c3 = pltpu.async_copy(table.at[ivf.at[pl.ds(NF * B, TL)]],
                              ri1.at[pl.ds(0, TL)], s3)
        c4 = pltpu.async_copy(table.at[jvf.at[pl.ds(NF * B, TL)]],
                              rj1.at[pl.ds(0, TL)], s4)
        c3.wait()
        c4.wait()
        w3 = pltpu.async_copy(ri1.at[pl.ds(0, TL)],
                              hi_hbm.at[pl.ds(base + NF * B, TL)], s3)
        w4 = pltpu.async_copy(rj1.at[pl.ds(0, TL)],
                              hj_hbm.at[pl.ds(base + NF * B, TL)], s4)
        w3.wait()
        w4.wait()

    return k(h1, ii, jj)


def _sc_agg(mji, mkj, sb, jid, kid):
    C = 6400
    KC = E // C
    TS = T // NS
    NV = TS // 16
    B = 64
    MAXB = (TS + 127) // 128 + 1
    RW = C // NS

    @functools.partial(
        pl.kernel, mesh=_mesh(), compiler_params=_sc_params,
        out_type=jax.ShapeDtypeStruct((E, D), jnp.float32),
        scratch_types=[
            pltpu.VMEM((TS,), jnp.int32),
            pltpu.VMEM((TS,), jnp.int32),
            pltpu.VMEM((MAXB, 128), jnp.int32),
            pltpu.VMEM((MAXB, 128), jnp.int32),
            pltpu.VMEM((2, B), jnp.int32),
            pltpu.VMEM((2, B), jnp.int32),
            pltpu.VMEM((B, D), jnp.float32),
            pltpu.VMEM((B, D), jnp.float32),
            pltpu.VMEM((B, D), jnp.float32),
            pltpu.VMEM((B, D), jnp.float32),
            pltpu.VMEM_SHARED((C + 8, D), jnp.float32),
            pltpu.SemaphoreType.DMA,
            pltpu.SemaphoreType.DMA,
            pltpu.SemaphoreType.DMA,
            pltpu.SemaphoreType.DMA,
            pltpu.SemaphoreType.DMA,
            pltpu.SemaphoreType.DMA,
        ],
    )
    def k(mji_hbm, mkj_hbm, sb_hbm, jid_hbm, kid_hbm, out_hbm,
          jv_ref, kv_ref, cbk, cbtd, stg_t, stg_d, rowsA0, rowsB0,
          rowsA1, rowsB1, acc, s1, s2, s3, s4, s5, s6):
        cid = lax.axis_index("c")
        sid = lax.axis_index("s")
        wid = sid * NC + cid
        tbase = sid * TS
        pltpu.sync_copy(jid_hbm.at[pl.ds(tbase, TS)], jv_ref)
        pltpu.sync_copy(kid_hbm.at[pl.ds(tbase, TS)], kv_ref)
        lanes = lax.iota(jnp.int32, 16)
        ck_lo = cid * 13
        ck_hi = jnp.where(cid == 0, 13, KC)

        def unpack(slot, b):
            row = lax.shift_right_logical(b, 1)
            colb = lax.bitwise_and(b, 1) * B
            for u in range(B // 16):
                val = cbtd[row, pl.ds(colb + u * 16, 16)]
                stg_t[slot, pl.ds(u * 16, 16)] = (
                    tbase + lax.shift_right_logical(val, 13))
                stg_d[slot, pl.ds(u * 16, 16)] = lax.bitwise_and(val, 8191)

        def issue(slot, b, rowsA, rowsB, sa, sb_):
            unpack(slot, b)
            row = lax.shift_right_logical(b, 1)
            colb = lax.bitwise_and(b, 1) * B
            ca = pltpu.async_copy(
                mkj_hbm.at[cbk.at[row, pl.ds(colb, B)]], rowsA, sa)
            cb = pltpu.async_copy(sb_hbm.at[stg_t.at[slot]], rowsB, sb_)
            return ca, cb

        def mul_sc(slot, rowsA, rowsB, sem):
            @pl.loop(0, B)
            def mul_row(r):
                for cc in range(0, D, 16):
                    rowsA[r, pl.ds(cc, 16)] = (rowsA[r, pl.ds(cc, 16)]
                                               * rowsB[r, pl.ds(cc, 16)])

            return pltpu.async_copy(rowsA, acc.at[stg_d.at[slot]], sem,
                                    add=True)

        def chunk(ck, _):
            cbase = ck * C
            cinit = pltpu.async_copy(mji_hbm.at[pl.ds(cbase + sid * RW, RW)],
                                     acc.at[pl.ds(sid * RW, RW)], s5)

            def scan_body(v, off):
                jx = jv_ref[pl.ds(v * 16, 16)]
                lj = jx - cbase
                msk = (lj >= 0) & (lj < C)
                ones = msk.astype(jnp.int32)
                tot = jnp.sum(ones)

                @pl.when(tot > 0)
                def store():
                    kx = kv_ref[pl.ds(v * 16, 16)]
                    inc = plsc.cumsum(ones)
                    pos = off + inc - 1
                    prow = lax.shift_right_logical(pos, 7)
                    pcol = lax.bitwise_and(pos, 127)
                    trel = v * 16 + lanes
                    packed = lax.bitwise_or(lax.shift_left(trel, 13), lj)
                    plsc.store_scatter(cbk, [prow, pcol], kx, mask=msk)
                    plsc.store_scatter(cbtd, [prow, pcol], packed, mask=msk)

                return off + tot

            off = lax.fori_loop(0, NV, scan_body, jnp.int32(0))
            nb = lax.shift_right_logical(off + B - 1, 6)
            pstart = lax.shift_right_logical(off, 4) * 16
            npv = lax.shift_right_logical(nb * B - pstart, 4)

            def pad_body(q, _):
                pos = pstart + q * 16 + lanes
                mskp = pos >= off
                prow = lax.shift_right_logical(pos, 7)
                pcol = lax.bitwise_and(pos, 127)
                padk = wid * 16 + lanes
                trel = lax.bitwise_and(lanes, 7)
                packed = lax.bitwise_or(lax.shift_left(trel, 13),
                                        C + trel)
                plsc.store_scatter(cbk, [prow, pcol], padk, mask=mskp)
                plsc.store_scatter(cbtd, [prow, pcol], packed, mask=mskp)
                return 0

            lax.fori_loop(0, npv, pad_body, 0)
            cinit.wait()
            plsc.subcore_barrier()

            def pair_body(bb, _):
                b0 = bb * 2
                b1 = b0 + 1
                c1, c2 = issue(0, b0, rowsA0, rowsB0, s1, s2)
                c3, c4 = issue(1, b1, rowsA1, rowsB1, s3, s4)
                c1.wait()
                c2.wait()
                w0 = mul_sc(0, rowsA0, rowsB0, s5)
                c3.wait()
                c4.wait()
                w1 = mul_sc(1, rowsA1, rowsB1, s6)
                w0.wait()
                w1.wait()
                return 0

            lax.fori_loop(0, lax.shift_right_logical(nb, 1), pair_body, 0)

            @pl.when(lax.bitwise_and(nb, 1) == 1)
            def tail():
                c1, c2 = issue(0, nb - 1, rowsA0, rowsB0, s1, s2)
                c1.wait()
                c2.wait()
                mul_sc(0, rowsA0, rowsB0, s5).wait()

            plsc.subcore_barrier()
            pltpu.sync_copy(acc.at[pl.ds(sid * RW, RW)],
                            out_hbm.at[pl.ds(cbase + sid * RW, RW)])
            plsc.subcore_barrier()
            return 0

        lax.fori_loop(ck_lo, ck_hi, chunk, 0)

    return k(mji, mkj, sb, jid, kid)


def _sc_nagg(m2, ro, iidx):
    C0 = 5120
    ES = E // NS
    NV = ES // 16
    B = 64
    MAXB = (ES + 127) // 128 + 1
    ZB = 16

    @functools.partial(
        pl.kernel, mesh=_mesh(), compiler_params=_sc_params,
        out_type=jax.ShapeDtypeStruct((N, D), jnp.float32),
        scratch_types=[
            pltpu.VMEM((ES,), jnp.int32),
            pltpu.VMEM((MAXB, 128), jnp.int32),
            pltpu.VMEM((MAXB, 128), jnp.int32),
            pltpu.VMEM((2, B), jnp.int32),
            pltpu.VMEM((B, D), jnp.float32),
            pltpu.VMEM((B, D), jnp.float32),
            pltpu.VMEM((B, D), jnp.float32),
            pltpu.VMEM((B, D), jnp.float32),
            pltpu.VMEM((ZB, D), jnp.float32),
            pltpu.VMEM_SHARED((C0 + 8, D), jnp.float32),
            pltpu.SemaphoreType.DMA,
            pltpu.SemaphoreType.DMA,
            pltpu.SemaphoreType.DMA,
            pltpu.SemaphoreType.DMA,
            pltpu.SemaphoreType.DMA,
            pltpu.SemaphoreType.DMA,
        ],
    )
    def k(m2_hbm, ro_hbm, i_hbm, out_hbm,
          iv_ref, cbe, cbd, stg_d, rowsA0, rowsB0, rowsA1, rowsB1, zbuf,
          acc, s1, s2, s3, s4, s5, s6):
        cid = lax.axis_index("c")
        sid = lax.axis_index("s")
        wid = sid * NC + cid
        ebase = sid * ES
        pltpu.sync_copy(i_hbm.at[pl.ds(ebase, ES)], iv_ref)
        lanes = lax.iota(jnp.int32, 16)
        nbase = cid * C0
        climit = jnp.where(cid == 0, C0, N - C0)
        zoff = jnp.where(cid == 0, sid * 320, sid * 304)
        zn = jnp.where(cid == 0, 320,
                       jnp.where(sid < 15, 304, 320))

        @pl.loop(0, ZB)
        def zrow(r):
            for cc in range(0, D, 16):
                zbuf[r, pl.ds(cc, 16)] = jnp.zeros((16,), jnp.float32)

        def zcopy(z, _):
            pltpu.async_copy(zbuf, acc.at[pl.ds(zoff + z * ZB, ZB)], s5)
            return 0

        lax.fori_loop(0, zn // ZB, zcopy, 0)

        def scan_body(v, off):
            ix = iv_ref[pl.ds(v * 16, 16)]
            lj = ix - nbase
            msk = (lj >= 0) & (lj < climit)
            ones = msk.astype(jnp.int32)
            tot = jnp.sum(ones)

            @pl.when(tot > 0)
            def store():
                inc = plsc.cumsum(ones)
                pos = off + inc - 1
                prow = lax.shift_right_logical(pos, 7)
                pcol = lax.bitwise_and(pos, 127)
                evec = ebase + v * 16 + lanes
                plsc.store_scatter(cbe, [prow, pcol], evec, mask=msk)
                plsc.store_scatter(cbd, [prow, pcol], lj, mask=msk)

            return off + tot

        off = lax.fori_loop(0, NV, scan_body, jnp.int32(0))
        nb = lax.shift_right_logical(off + B - 1, 6)
        pstart = lax.shift_right_logical(off, 4) * 16
        npv = lax.shift_right_logical(nb * B - pstart, 4)

        def pad_body(q, _):
            pos = pstart + q * 16 + lanes
            mskp = pos >= off
            prow = lax.shift_right_logical(pos, 7)
            pcol = lax.bitwise_and(pos, 127)
            pade = wid * 16 + lanes
            padd = C0 + lax.bitwise_and(lanes, 7)
            plsc.store_scatter(cbe, [prow, pcol], pade, mask=mskp)
            plsc.store_scatter(cbd, [prow, pcol], padd, mask=mskp)
            return 0

        lax.fori_loop(0, npv, pad_body, 0)

        def zdrain(z, _):
            pltpu.make_async_copy(zbuf, acc.at[pl.ds(zoff + z * ZB, ZB)],
                                  s5).wait()
            return 0

        lax.fori_loop(0, zn // ZB, zdrain, 0)
        plsc.subcore_barrier()

        def stage(slot, b):
            row = lax.shift_right_logical(b, 1)
            colb = lax.bitwise_and(b, 1) * B
            for u in range(B // 16):
                stg_d[slot, pl.ds(u * 16, 16)] = cbd[row,
                                                     pl.ds(colb + u * 16, 16)]

        def issue(slot, b, rowsA, rowsB, sa, sb_):
            stage(slot, b)
            row = lax.shift_right_logical(b, 1)
            colb = lax.bitwise_and(b, 1) * B
            ca = pltpu.async_copy(
                m2_hbm.at[cbe.at[row, pl.ds(colb, B)]], rowsA, sa)
            cb = pltpu.async_copy(
                ro_hbm.at[cbe.at[row, pl.ds(colb, B)]], rowsB, sb_)
            return ca, cb

        def mul_sc(slot, rowsA, rowsB, sem):
            @pl.loop(0, B)
            def mul_row(r):
                for cc in range(0, D, 16):
                    rowsA[r, pl.ds(cc, 16)] = (rowsA[r, pl.ds(cc, 16)]
                                               * rowsB[r, pl.ds(cc, 16)])

            return pltpu.async_copy(rowsA, acc.at[stg_d.at[slot]], sem,
                                    add=True)

        def pair_body(bb, _):
            b0 = bb * 2
            b1 = b0 + 1
            c1, c2 = issue(0, b0, rowsA0, rowsB0, s1, s2)
            c3, c4 = issue(1, b1, rowsA1, rowsB1, s3, s4)
            c1.wait()
            c2.wait()
            w0 = mul_sc(0, rowsA0, rowsB0, s5)
            c3.wait()
            c4.wait()
            w1 = mul_sc(1, rowsA1, rowsB1, s6)
            w0.wait()
            w1.wait()
            return 0

        lax.fori_loop(0, lax.shift_right_logical(nb, 1), pair_body, 0)

        @pl.when(lax.bitwise_and(nb, 1) == 1)
        def tail():
            c1, c2 = issue(0, nb - 1, rowsA0, rowsB0, s1, s2)
            c1.wait()
            c2.wait()
            mul_sc(0, rowsA0, rowsB0, s5).wait()

        plsc.subcore_barrier()

        @pl.when(zn == 320)
        def wout():
            pltpu.sync_copy(acc.at[pl.ds(zoff, 320)],
                            out_hbm.at[pl.ds(nbase + zoff, 320)])

        @pl.when(zn == 304)
        def woutt():
            pltpu.sync_copy(acc.at[pl.ds(zoff, 304)],
                            out_hbm.at[pl.ds(nbase + zoff, 304)])

    return k(m2, ro, iidx)


def kernel(h, rbf, sbf1, sbf2, idx_kj, idx_ji_1, idx_jj, idx_ji_2,
           edge_index, params):
    p = params
    i32 = jnp.int32
    ii = edge_index[1].astype(i32)
    jj = edge_index[0].astype(i32)
    idx_kj = idx_kj.astype(i32)
    idx_ji_1 = idx_ji_1.astype(i32)
    idx_jj = idx_jj.astype(i32)
    idx_ji_2 = idx_ji_2.astype(i32)

    def b2(name):
        return p[name].reshape(1, D)

    kW, jW = p['kj_W'], p['ji1_W']

    h1 = _tc_node_mlp(h, p['h_W'], b2('h_b'))
    hi, hj = _sc_gather2(h1, ii, jj)
    sb1, sb2, ro = _tc_sbro(sbf1, sbf2, rbf,
                            p['s1a_W'], b2('s1a_b'), p['s1b_W'], b2('s1b_b'),
                            p['s2a_W'], b2('s2a_b'), p['s2b_W'], b2('s2b_b'),
                            p['rbfo_W'])
    m_kj, m_ji1 = _tc_edge1(hi, hj, rbf,
                            kW[:D], kW[D:2 * D], kW[2 * D:], b2('kj_b'),
                            p['rbf1_W'],
                            jW[:D], jW[D:2 * D], jW[2 * D:], b2('ji1_b'))
    m1 = _sc_agg(m_ji1, m_kj, sb1, idx_ji_1, idx_kj)
    m_jj, m_ji2 = _tc_edge2(m1, rbf, p['jj_W'], b2('jj_b'), p['rbf2_W'],
                            p['ji2_W'], b2('ji2_b'))
    m2 = _sc_agg(m_ji2, m_jj, sb2, idx_ji_2, idx_jj)
    h2raw = _sc_nagg(m2, ro, ii)

    yWpad = jnp.pad(p['yW_W'], ((0, 0), (0, D - 1)))
    ybpad = jnp.pad(p['yW_b'], (0, D - 1)).reshape(1, D)
    ws = [p['r1a_W'], b2('r1a_b'), p['r1b_W'], b2('r1b_b'),
          p['h_W'], b2('h_b'),
          p['r2a_W'], b2('r2a_b'), p['r2b_W'], b2('r2b_b'),
          p['r3a_W'], b2('r3a_b'), p['r3b_W'], b2('r3b_b'),
          p['y1_W'], b2('y1_b'), p['y2_W'], b2('y2_b'),
          p['y3_W'], b2('y3_b'), yWpad, ybpad]
    h2, yfull = _tc_final(h2raw, h, ws)
    return (h2, yfull[:, :1])
